# Initial kernel scaffold; baseline (speedup 1.0000x reference)
import jax
import jax.numpy as jnp
from jax.experimental import pallas as pl


def kernel(fea_i, fea_last, xyz_i, xyz_last, batch, t_i, p_w1, p_b1, p_g1, p_be1, p_w2, p_b2, q_w, q_b, k_w, k_b, v_w, v_b, w_g1, w_be1, w_w, w_b, w_g2, w_be2):
    raise NotImplementedError("write your pallas kernel here")


# speedup vs baseline: 1.5917x; 1.5917x over previous
import functools

import jax
import jax.numpy as jnp
from jax import lax
from jax.experimental import pallas as pl
from jax.experimental.pallas import tpu as pltpu
from jax.experimental.pallas import tpu_sc as plsc

_N = 8192
_K = 16
_C = 256
_PH = 64
_T_LAST = 1.0
_EPS = 1e-5
_NK = _N * _K

_MASKVAL = 1e38

_RBP = 256


def _proj_body(fl_ref, fi_ref, qwt_ref, qb_ref, kwt_ref, kb_ref, vwt_ref, vb_ref,
               ql_ref, kv_ref):
    fl = fl_ref[...]
    fi = fi_ref[...]
    ql_ref[...] = jnp.dot(fl, qwt_ref[...], preferred_element_type=jnp.float32) + qb_ref[...]
    kv_ref[:, 0:_C] = jnp.dot(fi, kwt_ref[...], preferred_element_type=jnp.float32) + kb_ref[...]
    kv_ref[:, _C:2 * _C] = jnp.dot(fi, vwt_ref[...], preferred_element_type=jnp.float32) + vb_ref[...]


def _proj(fea_last, fea_i, q_wt, q_b, k_wt, k_b, v_wt, v_b):
    grid = (_N // _RBP,)
    return pl.pallas_call(
        _proj_body,
        grid=grid,
        in_specs=[
            pl.BlockSpec((_RBP, _C), lambda b: (b, 0)),
            pl.BlockSpec((_RBP, _C), lambda b: (b, 0)),
            pl.BlockSpec((_C, _C), lambda b: (0, 0)),
            pl.BlockSpec((1, _C), lambda b: (0, 0)),
            pl.BlockSpec((_C, _C), lambda b: (0, 0)),
            pl.BlockSpec((1, _C), lambda b: (0, 0)),
            pl.BlockSpec((_C, _C), lambda b: (0, 0)),
            pl.BlockSpec((1, _C), lambda b: (0, 0)),
        ],
        out_specs=[
            pl.BlockSpec((_RBP, _C), lambda b: (b, 0)),
            pl.BlockSpec((_RBP, 2 * _C), lambda b: (b, 0)),
        ],
        out_shape=[
            jax.ShapeDtypeStruct((_N, _C), jnp.float32),
            jax.ShapeDtypeStruct((_N, 2 * _C), jnp.float32),
        ],
    )(fea_last, fea_i, q_wt, q_b, k_wt, k_b, v_wt, v_b)


_RB = 128
_CT = 512
_NT = _N // _CT


def _knn_body(y_ref, by_ref, xt_ref, bx_ref, idx_ref, d_ref):
    y = y_ref[...]
    yb = y.astype(jnp.bfloat16)
    by = by_ref[...]
    big = jnp.int32(2 ** 30)
    yy = y[:, 0:1] * y[:, 0:1] + y[:, 1:2] * y[:, 1:2] + y[:, 2:3] * y[:, 2:3]

    def fill(t, c):
        sl = pl.ds(t * _CT, _CT)
        xs = xt_ref[:, sl]
        x0 = xs[0:1, :]
        x1 = xs[1:2, :]
        x2 = xs[2:3, :]
        xx = x0 * x0 + x1 * x1 + x2 * x2
        dot = jnp.dot(yb, xs.astype(jnp.bfloat16), preferred_element_type=jnp.float32)
        d = yy + xx - 2.0 * dot
        bx = bx_ref[:, sl]
        d_ref[:, sl] = jnp.where(bx != by, jnp.float32(_MASKVAL), d)
        return c

    lax.fori_loop(0, _NT, fill, 0)

    for k in range(_K):
        def tmin(t, m):
            tile = d_ref[:, pl.ds(t * _CT, _CT)]
            return jnp.minimum(m, jnp.min(tile, axis=1, keepdims=True))

        m = lax.fori_loop(0, _NT, tmin, jnp.full((_RB, 1), jnp.inf, jnp.float32))

        def tamin(t, a):
            tile = d_ref[:, pl.ds(t * _CT, _CT)]
            coli = lax.broadcasted_iota(jnp.int32, (_RB, _CT), 1) + t * _CT
            return jnp.minimum(a, jnp.min(jnp.where(tile <= m, coli, big), axis=1, keepdims=True))

        amin = lax.fori_loop(0, _NT, tamin, jnp.full((_RB, 1), big, jnp.int32))
        idx_ref[:, k:k + 1] = amin

        def tupd(t, c):
            sl = pl.ds(t * _CT, _CT)
            coli = lax.broadcasted_iota(jnp.int32, (_RB, _CT), 1) + t * _CT
            d_ref[:, sl] = jnp.where(coli == amin, jnp.float32(jnp.inf), d_ref[:, sl])
            return c

        lax.fori_loop(0, _NT, tupd, 0)


def _knn(xyz_last, by, xyz_i_t, bx):
    grid = (_N // _RB,)
    return pl.pallas_call(
        _knn_body,
        grid=grid,
        in_specs=[
            pl.BlockSpec((_RB, 8), lambda b: (b, 0)),
            pl.BlockSpec((_RB, 1), lambda b: (b, 0)),
            pl.BlockSpec((8, _N), lambda b: (0, 0)),
            pl.BlockSpec((1, _N), lambda b: (0, 0)),
        ],
        out_specs=pl.BlockSpec((_RB, _K), lambda b: (b, 0)),
        out_shape=jax.ShapeDtypeStruct((_N, _K), jnp.int32),
        scratch_shapes=[pltpu.VMEM((_RB, _N), jnp.float32)],
    )(xyz_last, by, xyz_i_t, bx)


_NW = 32
_BPW = _NK // _NW
_G = 128


_TW = 2 * _C + 128


def _gather(table, idx_flat):
    mesh = plsc.VectorSubcoreMesh(core_axis_name="c", subcore_axis_name="s")

    @functools.partial(
        pl.kernel,
        mesh=mesh,
        out_type=jax.ShapeDtypeStruct((_NK, _TW), jnp.float32),
        scratch_types=[
            pltpu.VMEM((_G,), jnp.int32),
            pltpu.VMEM((_G, _TW), jnp.float32),
            pltpu.SemaphoreType.DMA,
        ],
    )
    def gk(t_hbm, idx_hbm, o_hbm, idx_v, r1, sem):
        wid = lax.axis_index("s") * 2 + lax.axis_index("c")
        base = wid * _BPW

        def body(i, c):
            off = base + i * _G
            pltpu.sync_copy(idx_hbm.at[pl.ds(off, _G)], idx_v)
            pltpu.async_copy(t_hbm.at[idx_v], r1, sem).wait()
            pltpu.sync_copy(r1, o_hbm.at[pl.ds(off, _G)])
            return c

        lax.fori_loop(0, _BPW // _G, body, 0)

    return gk(table, idx_flat)


_FB = 512
_NP = _FB // _K


def _expand_mat():
    r = lax.broadcasted_iota(jnp.int32, (_FB, _NP), 0) // _K
    c = lax.broadcasted_iota(jnp.int32, (_FB, _NP), 1)
    return (r == c).astype(jnp.float32)


def _leaky(x):
    return jnp.where(x >= 0, x, 0.01 * x)


def _pe1stat_body(xg_ref, xl8_ref, pw1t_ref, pb1_ref, st_ref):
    e = _expand_mat()
    delta = xg_ref[:, 0:8] - jnp.dot(e, xl8_ref[...], preferred_element_type=jnp.float32)
    pe1 = jnp.dot(delta, pw1t_ref[...], preferred_element_type=jnp.float32) + pb1_ref[...]
    st_ref[0:1, 0:1, 0:_PH] = jnp.sum(pe1, axis=0, keepdims=True).reshape(1, 1, _PH)
    st_ref[0:1, 0:1, _PH:2 * _PH] = jnp.sum(pe1 * pe1, axis=0, keepdims=True).reshape(1, 1, _PH)


def _pe1stat(g, xl8, pw1t8, pb1):
    grid = (_NK // _FB,)
    return pl.pallas_call(
        _pe1stat_body,
        grid=grid,
        in_specs=[
            pl.BlockSpec((_FB, 128), lambda b: (b, 4)),
            pl.BlockSpec((_NP, 8), lambda b: (b, 0)),
            pl.BlockSpec((8, _PH), lambda b: (0, 0)),
            pl.BlockSpec((1, _PH), lambda b: (0, 0)),
        ],
        out_specs=pl.BlockSpec((1, 1, 2 * _PH), lambda b: (b, 0, 0)),
        out_shape=jax.ShapeDtypeStruct((_NK // _FB, 1, 2 * _PH), jnp.float32),
    )(g, xl8, pw1t8, pb1)


def _passB_body(xg_ref, kg_ref, vg_ref, ql_ref, xl8_ref, pw1t_ref, pb1_ref,
                a1_ref, c1_ref, pw2t_ref, pb2_ref,
                wpre_ref, vout_ref, st_ref):
    e = _expand_mat()
    delta = xg_ref[:, 0:8] - jnp.dot(e, xl8_ref[...], preferred_element_type=jnp.float32)
    pe1 = jnp.dot(delta, pw1t_ref[...], preferred_element_type=jnp.float32) + pb1_ref[...]
    pe1 = _leaky(pe1 * a1_ref[...] + c1_ref[...])
    pe2 = jnp.dot(pe1, pw2t_ref[...], preferred_element_type=jnp.float32) + pb2_ref[...]
    qlr = jnp.dot(e, ql_ref[...], preferred_element_type=jnp.float32)
    wpre = qlr - kg_ref[...] + pe2
    wpre_ref[...] = wpre
    vout_ref[...] = vg_ref[...] + pe2
    st_ref[0:1, 0:1, 0:_C] = jnp.sum(wpre, axis=0, keepdims=True).reshape(1, 1, _C)
    st_ref[0:1, 0:1, _C:2 * _C] = jnp.sum(wpre * wpre, axis=0, keepdims=True).reshape(1, 1, _C)


def _passB(g, ql, xl8, pw1t8, pb1, a1, c1, pw2t, pb2):
    grid = (_NK // _FB,)
    return pl.pallas_call(
        _passB_body,
        grid=grid,
        in_specs=[
            pl.BlockSpec((_FB, 128), lambda b: (b, 4)),
            pl.BlockSpec((_FB, _C), lambda b: (b, 0)),
            pl.BlockSpec((_FB, _C), lambda b: (b, 1)),
            pl.BlockSpec((_NP, _C), lambda b: (b, 0)),
            pl.BlockSpec((_NP, 8), lambda b: (b, 0)),
            pl.BlockSpec((8, _PH), lambda b: (0, 0)),
            pl.BlockSpec((1, _PH), lambda b: (0, 0)),
            pl.BlockSpec((1, _PH), lambda b: (0, 0)),
            pl.BlockSpec((1, _PH), lambda b: (0, 0)),
            pl.BlockSpec((_PH, _C), lambda b: (0, 0)),
            pl.BlockSpec((1, _C), lambda b: (0, 0)),
        ],
        out_specs=[
            pl.BlockSpec((_FB, _C), lambda b: (b, 0)),
            pl.BlockSpec((_FB, _C), lambda b: (b, 0)),
            pl.BlockSpec((1, 1, 2 * _C), lambda b: (b, 0, 0)),
        ],
        out_shape=[
            jax.ShapeDtypeStruct((_NK, _C), jnp.float32),
            jax.ShapeDtypeStruct((_NK, _C), jnp.float32),
            jax.ShapeDtypeStruct((_NK // _FB, 1, 2 * _C), jnp.float32),
        ],
    )(g, g, g, ql, xl8, pw1t8, pb1, a1, c1, pw2t, pb2)


def _passC_body(wpre_ref, ag_ref, cg_ref, wwt_ref, wb_ref, h_ref, st_ref):
    s = _leaky(wpre_ref[...] * ag_ref[...] + cg_ref[...])
    h = jnp.dot(s, wwt_ref[...], preferred_element_type=jnp.float32) + wb_ref[...]
    h_ref[...] = h
    st_ref[0:1, 0:1, 0:_C] = jnp.sum(h, axis=0, keepdims=True).reshape(1, 1, _C)
    st_ref[0:1, 0:1, _C:2 * _C] = jnp.sum(h * h, axis=0, keepdims=True).reshape(1, 1, _C)


def _passC(wpre, ag1, cg1, wwt, wb):
    grid = (_NK // _FB,)
    return pl.pallas_call(
        _passC_body,
        grid=grid,
        in_specs=[
            pl.BlockSpec((_FB, _C), lambda b: (b, 0)),
            pl.BlockSpec((1, _C), lambda b: (0, 0)),
            pl.BlockSpec((1, _C), lambda b: (0, 0)),
            pl.BlockSpec((_C, _C), lambda b: (0, 0)),
            pl.BlockSpec((1, _C), lambda b: (0, 0)),
        ],
        out_specs=[
            pl.BlockSpec((_FB, _C), lambda b: (b, 0)),
            pl.BlockSpec((1, 1, 2 * _C), lambda b: (b, 0, 0)),
        ],
        out_shape=[
            jax.ShapeDtypeStruct((_NK, _C), jnp.float32),
            jax.ShapeDtypeStruct((_NK // _FB, 1, 2 * _C), jnp.float32),
        ],
    )(wpre, ag1, cg1, wwt, wb)


def _tree_red(x, op):
    w = _K
    while w > 1:
        h = w // 2
        x = op(x[:, 0:h], x[:, h:w])
        w = h
    return x


def _passD_body(h_ref, v_ref, ag_ref, cg_ref, out_ref):
    s = _leaky(h_ref[...] * ag_ref[...] + cg_ref[...])
    m = _tree_red(s, jnp.maximum)
    ex = jnp.exp(s - m)
    den = _tree_red(ex, jnp.add)
    w = ex / den
    out_ref[...] = _tree_red(w * v_ref[...], jnp.add)


def _passD(h3, v3, ag2, cg2):
    grid = (_N // _NP,)
    return pl.pallas_call(
        _passD_body,
        grid=grid,
        in_specs=[
            pl.BlockSpec((_NP, _K, _C), lambda b: (b, 0, 0)),
            pl.BlockSpec((_NP, _K, _C), lambda b: (b, 0, 0)),
            pl.BlockSpec((1, 1, _C), lambda b: (0, 0, 0)),
            pl.BlockSpec((1, 1, _C), lambda b: (0, 0, 0)),
        ],
        out_specs=pl.BlockSpec((_NP, 1, _C), lambda b: (b, 0, 0)),
        out_shape=jax.ShapeDtypeStruct((_N, 1, _C), jnp.float32),
    )(h3, v3, ag2, cg2)


def _bn_ab(stats3, width, gamma, beta):
    stats = stats3.reshape(-1, 2 * width)
    s1 = jnp.sum(stats[:, 0:width], axis=0)
    s2 = jnp.sum(stats[:, width:2 * width], axis=0)
    mean = s1 / _NK
    var = s2 / _NK - mean * mean
    a = gamma / jnp.sqrt(var + _EPS)
    b = beta - mean * a
    return a.reshape(1, width), b.reshape(1, width)


def kernel(fea_i, fea_last, xyz_i, xyz_last, batch, t_i,
           p_w1, p_b1, p_g1, p_be1, p_w2, p_b2,
           q_w, q_b, k_w, k_b, v_w, v_b,
           w_g1, w_be1, w_w, w_b, w_g2, w_be2):
    f32 = jnp.float32
    t_i = jnp.asarray(t_i, f32)

    ql, kv = _proj(fea_last, fea_i,
                   q_w.T, q_b.reshape(1, _C), k_w.T, k_b.reshape(1, _C),
                   v_w.T, v_b.reshape(1, _C))

    by = batch.reshape(_N, 1)
    bx = batch.reshape(1, _N)
    y8 = jnp.concatenate([xyz_last, jnp.zeros((_N, 5), f32)], axis=1)
    xt8 = jnp.concatenate([xyz_i.T, jnp.zeros((5, _N), f32)], axis=0)
    idx = _knn(y8, by, xt8, bx)
    idx_flat = idx.reshape(_NK)

    table = jnp.concatenate(
        [kv, xyz_i, jnp.full((_N, 1), t_i, f32), jnp.zeros((_N, 124), f32)], axis=1)
    g = _gather(table, idx_flat)

    xl8 = jnp.concatenate(
        [xyz_last, jnp.full((_N, 1), _T_LAST, f32), jnp.zeros((_N, 4), f32)], axis=1)
    pw1t8 = jnp.concatenate([p_w1.T, jnp.zeros((4, _PH), f32)], axis=0)
    pb1 = p_b1.reshape(1, _PH)

    st1 = _pe1stat(g, xl8, pw1t8, pb1)
    a1, c1 = _bn_ab(st1, _PH, p_g1, p_be1)

    wpre, vout, st2 = _passB(g, ql, xl8, pw1t8, pb1, a1, c1,
                             p_w2.T, p_b2.reshape(1, _C))
    ag1, cg1 = _bn_ab(st2, _C, w_g1, w_be1)

    h, st3 = _passC(wpre, ag1, cg1, w_w.T, w_b.reshape(1, _C))
    ag2, cg2 = _bn_ab(st3, _C, w_g2, w_be2)

    out3 = _passD(h.reshape(_N, _K, _C), vout.reshape(_N, _K, _C),
                  ag2.reshape(1, 1, _C), cg2.reshape(1, 1, _C))
    return out3.reshape(_N, _C)


# speedup vs baseline: 4.6975x; 2.9513x over previous
import functools

import jax
import jax.numpy as jnp
from jax import lax
from jax.experimental import pallas as pl
from jax.experimental.pallas import tpu as pltpu
from jax.experimental.pallas import tpu_sc as plsc

_N = 8192
_K = 16
_C = 256
_PH = 64
_T_LAST = 1.0
_EPS = 1e-5
_NK = _N * _K

_MASKVAL = 1e38

_RBP = 256


def _proj_body(fl_ref, fi_ref, qwt_ref, qb_ref, kwt_ref, kb_ref, vwt_ref, vb_ref,
               ql_ref, kv_ref):
    fl = fl_ref[...]
    fi = fi_ref[...]
    ql_ref[...] = jnp.dot(fl, qwt_ref[...], preferred_element_type=jnp.float32) + qb_ref[...]
    kv_ref[:, 0:_C] = jnp.dot(fi, kwt_ref[...], preferred_element_type=jnp.float32) + kb_ref[...]
    kv_ref[:, _C:2 * _C] = jnp.dot(fi, vwt_ref[...], preferred_element_type=jnp.float32) + vb_ref[...]


def _proj(fea_last, fea_i, q_wt, q_b, k_wt, k_b, v_wt, v_b):
    grid = (_N // _RBP,)
    return pl.pallas_call(
        _proj_body,
        grid=grid,
        in_specs=[
            pl.BlockSpec((_RBP, _C), lambda b: (b, 0)),
            pl.BlockSpec((_RBP, _C), lambda b: (b, 0)),
            pl.BlockSpec((_C, _C), lambda b: (0, 0)),
            pl.BlockSpec((1, _C), lambda b: (0, 0)),
            pl.BlockSpec((_C, _C), lambda b: (0, 0)),
            pl.BlockSpec((1, _C), lambda b: (0, 0)),
            pl.BlockSpec((_C, _C), lambda b: (0, 0)),
            pl.BlockSpec((1, _C), lambda b: (0, 0)),
        ],
        out_specs=[
            pl.BlockSpec((_RBP, _C), lambda b: (b, 0)),
            pl.BlockSpec((_RBP, 2 * _C), lambda b: (b, 0)),
        ],
        out_shape=[
            jax.ShapeDtypeStruct((_N, _C), jnp.float32),
            jax.ShapeDtypeStruct((_N, 2 * _C), jnp.float32),
        ],
    )(fea_last, fea_i, q_wt, q_b, k_wt, k_b, v_wt, v_b)


_RB = 128
_CT = 512
_NT = _N // _CT


def _knn_body(tlo_ref, thi_ref, y_ref, by_ref, xt_ref, bx_ref, idx_ref, d_ref):
    pid = pl.program_id(0)
    tlo = tlo_ref[pid]
    thi = thi_ref[pid]
    y = y_ref[...]
    yb = y.astype(jnp.bfloat16)
    by = by_ref[...]
    big = jnp.int32(2 ** 30)
    inf = jnp.float32(jnp.inf)
    yy = y[:, 0:1] * y[:, 0:1] + y[:, 1:2] * y[:, 1:2] + y[:, 2:3] * y[:, 2:3]

    am = jnp.full((_RB, 1), -1, jnp.int32)
    for k in range(_K):
        am_prev = am

        def step(t, carry, am_prev=am_prev, first=(k == 0), last=(k == _K - 1)):
            m, am = carry
            sl = pl.ds(t * _CT, _CT)
            coli = lax.broadcasted_iota(jnp.int32, (_RB, _CT), 1) + t * _CT
            if first:
                xs = xt_ref[:, sl]
                x0 = xs[0:1, :]
                x1 = xs[1:2, :]
                x2 = xs[2:3, :]
                xx = x0 * x0 + x1 * x1 + x2 * x2
                dot = jnp.dot(yb, xs.astype(jnp.bfloat16),
                              preferred_element_type=jnp.float32)
                tile = yy + xx - 2.0 * dot
                bx = bx_ref[:, sl]
                tile = jnp.where(bx != by, jnp.float32(_MASKVAL), tile)
            else:
                tile = jnp.where(coli == am_prev, inf, d_ref[:, sl])
            if not last:
                d_ref[:, sl] = tile
            tmin = jnp.min(tile, axis=1, keepdims=True)
            tam = jnp.min(jnp.where(tile <= tmin, coli, big), axis=1, keepdims=True)
            upd = tmin < m
            return (jnp.where(upd, tmin, m), jnp.where(upd, tam, am))

        m, am = lax.fori_loop(
            tlo, thi, step,
            (jnp.full((_RB, 1), inf, jnp.float32), jnp.full((_RB, 1), big, jnp.int32)))
        idx_ref[:, k:k + 1] = am


def _knn(xyz_last, by, xyz_i_t, bx, tlo, thi):
    grid_spec = pltpu.PrefetchScalarGridSpec(
        num_scalar_prefetch=2,
        grid=(_N // _RB,),
        in_specs=[
            pl.BlockSpec((_RB, 8), lambda b, *_: (b, 0)),
            pl.BlockSpec((_RB, 1), lambda b, *_: (b, 0)),
            pl.BlockSpec((8, _N), lambda b, *_: (0, 0)),
            pl.BlockSpec((1, _N), lambda b, *_: (0, 0)),
        ],
        out_specs=pl.BlockSpec((_RB, _K), lambda b, *_: (b, 0)),
        scratch_shapes=[pltpu.VMEM((_RB, _N), jnp.float32)],
    )
    return pl.pallas_call(
        _knn_body,
        grid_spec=grid_spec,
        out_shape=jax.ShapeDtypeStruct((_N, _K), jnp.int32),
    )(tlo, thi, xyz_last, by, xyz_i_t, bx)


_NW = 32
_BPW = _NK // _NW
_G = 128


_TW = 2 * _C + 128


def _gather(table, idx_flat):
    mesh = plsc.VectorSubcoreMesh(core_axis_name="c", subcore_axis_name="s")

    @functools.partial(
        pl.kernel,
        mesh=mesh,
        out_type=jax.ShapeDtypeStruct((_NK, _TW), jnp.float32),
        scratch_types=[
            pltpu.VMEM((_G,), jnp.int32),
            pltpu.VMEM((_G, _TW), jnp.float32),
            pltpu.SemaphoreType.DMA,
        ],
    )
    def gk(t_hbm, idx_hbm, o_hbm, idx_v, r1, sem):
        wid = lax.axis_index("s") * 2 + lax.axis_index("c")
        base = wid * _BPW

        def body(i, c):
            off = base + i * _G
            pltpu.sync_copy(idx_hbm.at[pl.ds(off, _G)], idx_v)
            pltpu.async_copy(t_hbm.at[idx_v], r1, sem).wait()
            pltpu.sync_copy(r1, o_hbm.at[pl.ds(off, _G)])
            return c

        lax.fori_loop(0, _BPW // _G, body, 0)

    return gk(table, idx_flat)


_FB = 512
_NP = _FB // _K


def _expand_mat():
    r = lax.broadcasted_iota(jnp.int32, (_FB, _NP), 0) // _K
    c = lax.broadcasted_iota(jnp.int32, (_FB, _NP), 1)
    return (r == c).astype(jnp.float32)


def _leaky(x):
    return jnp.where(x >= 0, x, 0.01 * x)


def _pe1stat_body(xg_ref, xl8_ref, pw1t_ref, pb1_ref, st_ref):
    e = _expand_mat()
    delta = xg_ref[:, 0:8] - jnp.dot(e, xl8_ref[...], preferred_element_type=jnp.float32)
    pe1 = jnp.dot(delta, pw1t_ref[...], preferred_element_type=jnp.float32) + pb1_ref[...]
    st_ref[0:1, 0:1, 0:_PH] = jnp.sum(pe1, axis=0, keepdims=True).reshape(1, 1, _PH)
    st_ref[0:1, 0:1, _PH:2 * _PH] = jnp.sum(pe1 * pe1, axis=0, keepdims=True).reshape(1, 1, _PH)


def _pe1stat(g, xl8, pw1t8, pb1):
    grid = (_NK // _FB,)
    return pl.pallas_call(
        _pe1stat_body,
        grid=grid,
        in_specs=[
            pl.BlockSpec((_FB, 128), lambda b: (b, 4)),
            pl.BlockSpec((_NP, 8), lambda b: (b, 0)),
            pl.BlockSpec((8, _PH), lambda b: (0, 0)),
            pl.BlockSpec((1, _PH), lambda b: (0, 0)),
        ],
        out_specs=pl.BlockSpec((1, 1, 2 * _PH), lambda b: (b, 0, 0)),
        out_shape=jax.ShapeDtypeStruct((_NK // _FB, 1, 2 * _PH), jnp.float32),
    )(g, xl8, pw1t8, pb1)


def _passB_body(xg_ref, kg_ref, vg_ref, ql_ref, xl8_ref, pw1t_ref, pb1_ref,
                a1_ref, c1_ref, pw2t_ref, pb2_ref,
                wpre_ref, vout_ref, st_ref):
    e = _expand_mat()
    delta = xg_ref[:, 0:8] - jnp.dot(e, xl8_ref[...], preferred_element_type=jnp.float32)
    pe1 = jnp.dot(delta, pw1t_ref[...], preferred_element_type=jnp.float32) + pb1_ref[...]
    pe1 = _leaky(pe1 * a1_ref[...] + c1_ref[...])
    pe2 = jnp.dot(pe1, pw2t_ref[...], preferred_element_type=jnp.float32) + pb2_ref[...]
    qlr = jnp.dot(e, ql_ref[...], preferred_element_type=jnp.float32)
    wpre = qlr - kg_ref[...] + pe2
    wpre_ref[...] = wpre
    vout_ref[...] = vg_ref[...] + pe2
    st_ref[0:1, 0:1, 0:_C] = jnp.sum(wpre, axis=0, keepdims=True).reshape(1, 1, _C)
    st_ref[0:1, 0:1, _C:2 * _C] = jnp.sum(wpre * wpre, axis=0, keepdims=True).reshape(1, 1, _C)


def _passB(g, ql, xl8, pw1t8, pb1, a1, c1, pw2t, pb2):
    grid = (_NK // _FB,)
    return pl.pallas_call(
        _passB_body,
        grid=grid,
        in_specs=[
            pl.BlockSpec((_FB, 128), lambda b: (b, 4)),
            pl.BlockSpec((_FB, _C), lambda b: (b, 0)),
            pl.BlockSpec((_FB, _C), lambda b: (b, 1)),
            pl.BlockSpec((_NP, _C), lambda b: (b, 0)),
            pl.BlockSpec((_NP, 8), lambda b: (b, 0)),
            pl.BlockSpec((8, _PH), lambda b: (0, 0)),
            pl.BlockSpec((1, _PH), lambda b: (0, 0)),
            pl.BlockSpec((1, _PH), lambda b: (0, 0)),
            pl.BlockSpec((1, _PH), lambda b: (0, 0)),
            pl.BlockSpec((_PH, _C), lambda b: (0, 0)),
            pl.BlockSpec((1, _C), lambda b: (0, 0)),
        ],
        out_specs=[
            pl.BlockSpec((_FB, _C), lambda b: (b, 0)),
            pl.BlockSpec((_FB, _C), lambda b: (b, 0)),
            pl.BlockSpec((1, 1, 2 * _C), lambda b: (b, 0, 0)),
        ],
        out_shape=[
            jax.ShapeDtypeStruct((_NK, _C), jnp.float32),
            jax.ShapeDtypeStruct((_NK, _C), jnp.float32),
            jax.ShapeDtypeStruct((_NK // _FB, 1, 2 * _C), jnp.float32),
        ],
    )(g, g, g, ql, xl8, pw1t8, pb1, a1, c1, pw2t, pb2)


def _passC_body(wpre_ref, ag_ref, cg_ref, wwt_ref, wb_ref, h_ref, st_ref):
    s = _leaky(wpre_ref[...] * ag_ref[...] + cg_ref[...])
    h = jnp.dot(s, wwt_ref[...], preferred_element_type=jnp.float32) + wb_ref[...]
    h_ref[...] = h
    st_ref[0:1, 0:1, 0:_C] = jnp.sum(h, axis=0, keepdims=True).reshape(1, 1, _C)
    st_ref[0:1, 0:1, _C:2 * _C] = jnp.sum(h * h, axis=0, keepdims=True).reshape(1, 1, _C)


def _passC(wpre, ag1, cg1, wwt, wb):
    grid = (_NK // _FB,)
    return pl.pallas_call(
        _passC_body,
        grid=grid,
        in_specs=[
            pl.BlockSpec((_FB, _C), lambda b: (b, 0)),
            pl.BlockSpec((1, _C), lambda b: (0, 0)),
            pl.BlockSpec((1, _C), lambda b: (0, 0)),
            pl.BlockSpec((_C, _C), lambda b: (0, 0)),
            pl.BlockSpec((1, _C), lambda b: (0, 0)),
        ],
        out_specs=[
            pl.BlockSpec((_FB, _C), lambda b: (b, 0)),
            pl.BlockSpec((1, 1, 2 * _C), lambda b: (b, 0, 0)),
        ],
        out_shape=[
            jax.ShapeDtypeStruct((_NK, _C), jnp.float32),
            jax.ShapeDtypeStruct((_NK // _FB, 1, 2 * _C), jnp.float32),
        ],
    )(wpre, ag1, cg1, wwt, wb)


def _tree_red(x, op):
    w = _K
    while w > 1:
        h = w // 2
        x = op(x[:, 0:h], x[:, h:w])
        w = h
    return x


def _passD_body(h_ref, v_ref, ag_ref, cg_ref, out_ref):
    s = _leaky(h_ref[...] * ag_ref[...] + cg_ref[...])
    m = _tree_red(s, jnp.maximum)
    ex = jnp.exp(s - m)
    den = _tree_red(ex, jnp.add)
    w = ex / den
    out_ref[...] = _tree_red(w * v_ref[...], jnp.add)


def _passD(h3, v3, ag2, cg2):
    grid = (_N // _NP,)
    return pl.pallas_call(
        _passD_body,
        grid=grid,
        in_specs=[
            pl.BlockSpec((_NP, _K, _C), lambda b: (b, 0, 0)),
            pl.BlockSpec((_NP, _K, _C), lambda b: (b, 0, 0)),
            pl.BlockSpec((1, 1, _C), lambda b: (0, 0, 0)),
            pl.BlockSpec((1, 1, _C), lambda b: (0, 0, 0)),
        ],
        out_specs=pl.BlockSpec((_NP, 1, _C), lambda b: (b, 0, 0)),
        out_shape=jax.ShapeDtypeStruct((_N, 1, _C), jnp.float32),
    )(h3, v3, ag2, cg2)


def _bn_ab(stats3, width, gamma, beta):
    stats = stats3.reshape(-1, 2 * width)
    s1 = jnp.sum(stats[:, 0:width], axis=0)
    s2 = jnp.sum(stats[:, width:2 * width], axis=0)
    mean = s1 / _NK
    var = s2 / _NK - mean * mean
    a = gamma / jnp.sqrt(var + _EPS)
    b = beta - mean * a
    return a.reshape(1, width), b.reshape(1, width)


def kernel(fea_i, fea_last, xyz_i, xyz_last, batch, t_i,
           p_w1, p_b1, p_g1, p_be1, p_w2, p_b2,
           q_w, q_b, k_w, k_b, v_w, v_b,
           w_g1, w_be1, w_w, w_b, w_g2, w_be2):
    f32 = jnp.float32
    t_i = jnp.asarray(t_i, f32)

    ql, kv = _proj(fea_last, fea_i,
                   q_w.T, q_b.reshape(1, _C), k_w.T, k_b.reshape(1, _C),
                   v_w.T, v_b.reshape(1, _C))

    by = batch.reshape(_N, 1)
    bx = batch.reshape(1, _N)
    y8 = jnp.concatenate([xyz_last, jnp.zeros((_N, 5), f32)], axis=1)
    xt8 = jnp.concatenate([xyz_i.T, jnp.zeros((5, _N), f32)], axis=0)
    bounds = jnp.searchsorted(batch, jnp.arange(9, dtype=jnp.int32),
                              side="left").astype(jnp.int32)
    bf = batch[0::_RB]
    bl = batch[_RB - 1::_RB]
    tlo = (bounds[bf] // _CT).astype(jnp.int32)
    thi = ((bounds[bl + 1] + _CT - 1) // _CT).astype(jnp.int32)
    idx = _knn(y8, by, xt8, bx, tlo, thi)
    idx_flat = idx.reshape(_NK)

    table = jnp.concatenate(
        [kv, xyz_i, jnp.full((_N, 1), t_i, f32), jnp.zeros((_N, 124), f32)], axis=1)
    g = _gather(table, idx_flat)

    xl8 = jnp.concatenate(
        [xyz_last, jnp.full((_N, 1), _T_LAST, f32), jnp.zeros((_N, 4), f32)], axis=1)
    pw1t8 = jnp.concatenate([p_w1.T, jnp.zeros((4, _PH), f32)], axis=0)
    pb1 = p_b1.reshape(1, _PH)

    st1 = _pe1stat(g, xl8, pw1t8, pb1)
    a1, c1 = _bn_ab(st1, _PH, p_g1, p_be1)

    wpre, vout, st2 = _passB(g, ql, xl8, pw1t8, pb1, a1, c1,
                             p_w2.T, p_b2.reshape(1, _C))
    ag1, cg1 = _bn_ab(st2, _C, w_g1, w_be1)

    h, st3 = _passC(wpre, ag1, cg1, w_w.T, w_b.reshape(1, _C))
    ag2, cg2 = _bn_ab(st3, _C, w_g2, w_be2)

    out3 = _passD(h.reshape(_N, _K, _C), vout.reshape(_N, _K, _C),
                  ag2.reshape(1, 1, _C), cg2.reshape(1, 1, _C))
    return out3.reshape(_N, _C)


# speedup vs baseline: 4.9540x; 1.0546x over previous
import functools

import jax
import jax.numpy as jnp
from jax import lax
from jax.experimental import pallas as pl
from jax.experimental.pallas import tpu as pltpu
from jax.experimental.pallas import tpu_sc as plsc

_N = 8192
_K = 16
_C = 256
_PH = 64
_T_LAST = 1.0
_EPS = 1e-5
_NK = _N * _K

_MASKVAL = 1e38

_RBP = 256


def _proj_body(fl_ref, fi_ref, qwt_ref, qb_ref, kwt_ref, kb_ref, vwt_ref, vb_ref,
               ql_ref, kv_ref):
    fl = fl_ref[...].astype(jnp.bfloat16)
    fi = fi_ref[...].astype(jnp.bfloat16)
    ql_ref[...] = jnp.dot(fl, qwt_ref[...].astype(jnp.bfloat16),
                          preferred_element_type=jnp.float32) + qb_ref[...]
    kv_ref[:, 0:_C] = jnp.dot(fi, kwt_ref[...].astype(jnp.bfloat16),
                              preferred_element_type=jnp.float32) + kb_ref[...]
    kv_ref[:, _C:2 * _C] = jnp.dot(fi, vwt_ref[...].astype(jnp.bfloat16),
                                   preferred_element_type=jnp.float32) + vb_ref[...]


def _proj(fea_last, fea_i, q_wt, q_b, k_wt, k_b, v_wt, v_b):
    grid = (_N // _RBP,)
    return pl.pallas_call(
        _proj_body,
        grid=grid,
        in_specs=[
            pl.BlockSpec((_RBP, _C), lambda b: (b, 0)),
            pl.BlockSpec((_RBP, _C), lambda b: (b, 0)),
            pl.BlockSpec((_C, _C), lambda b: (0, 0)),
            pl.BlockSpec((1, _C), lambda b: (0, 0)),
            pl.BlockSpec((_C, _C), lambda b: (0, 0)),
            pl.BlockSpec((1, _C), lambda b: (0, 0)),
            pl.BlockSpec((_C, _C), lambda b: (0, 0)),
            pl.BlockSpec((1, _C), lambda b: (0, 0)),
        ],
        out_specs=[
            pl.BlockSpec((_RBP, _C), lambda b: (b, 0)),
            pl.BlockSpec((_RBP, 2 * _C), lambda b: (b, 0)),
        ],
        out_shape=[
            jax.ShapeDtypeStruct((_N, _C), jnp.float32),
            jax.ShapeDtypeStruct((_N, 2 * _C), jnp.float32),
        ],
    )(fea_last, fea_i, q_wt, q_b, k_wt, k_b, v_wt, v_b)


_RB = 128
_CT = 512
_NT = _N // _CT


def _knn_body(tlo_ref, thi_ref, y_ref, by_ref, xt_ref, bx_ref, idx_ref, d_ref):
    pid = pl.program_id(0)
    tlo = tlo_ref[pid]
    thi = thi_ref[pid]
    y = y_ref[...]
    yb = y.astype(jnp.bfloat16)
    by = by_ref[...]
    big = jnp.int32(2 ** 30)
    inf = jnp.float32(jnp.inf)
    yy = y[:, 0:1] * y[:, 0:1] + y[:, 1:2] * y[:, 1:2] + y[:, 2:3] * y[:, 2:3]

    am = jnp.full((_RB, 1), -1, jnp.int32)
    for k in range(_K):
        am_prev = am

        def step(t, carry, am_prev=am_prev, first=(k == 0), last=(k == _K - 1)):
            m, am = carry
            sl = pl.ds(t * _CT, _CT)
            coli = lax.broadcasted_iota(jnp.int32, (_RB, _CT), 1) + t * _CT
            if first:
                xs = xt_ref[:, sl]
                x0 = xs[0:1, :]
                x1 = xs[1:2, :]
                x2 = xs[2:3, :]
                xx = x0 * x0 + x1 * x1 + x2 * x2
                dot = jnp.dot(yb, xs.astype(jnp.bfloat16),
                              preferred_element_type=jnp.float32)
                tile = yy + xx - 2.0 * dot
                bx = bx_ref[:, sl]
                tile = jnp.where(bx != by, jnp.float32(_MASKVAL), tile)
            else:
                tile = jnp.where(coli == am_prev, inf, d_ref[:, sl])
            if not last:
                d_ref[:, sl] = tile
            tmin = jnp.min(tile, axis=1, keepdims=True)
            tam = jnp.min(jnp.where(tile <= tmin, coli, big), axis=1, keepdims=True)
            upd = tmin < m
            return (jnp.where(upd, tmin, m), jnp.where(upd, tam, am))

        m, am = lax.fori_loop(
            tlo, thi, step,
            (jnp.full((_RB, 1), inf, jnp.float32), jnp.full((_RB, 1), big, jnp.int32)))
        idx_ref[:, k:k + 1] = am


def _knn(xyz_last, by, xyz_i_t, bx, tlo, thi):
    grid_spec = pltpu.PrefetchScalarGridSpec(
        num_scalar_prefetch=2,
        grid=(_N // _RB,),
        in_specs=[
            pl.BlockSpec((_RB, 8), lambda b, *_: (b, 0)),
            pl.BlockSpec((_RB, 1), lambda b, *_: (b, 0)),
            pl.BlockSpec((8, _N), lambda b, *_: (0, 0)),
            pl.BlockSpec((1, _N), lambda b, *_: (0, 0)),
        ],
        out_specs=pl.BlockSpec((_RB, _K), lambda b, *_: (b, 0)),
        scratch_shapes=[pltpu.VMEM((_RB, _N), jnp.float32)],
    )
    return pl.pallas_call(
        _knn_body,
        grid_spec=grid_spec,
        out_shape=jax.ShapeDtypeStruct((_N, _K), jnp.int32),
    )(tlo, thi, xyz_last, by, xyz_i_t, bx)


_NW = 32
_BPW = _NK // _NW
_G = 128


_TW = 2 * _C + 128


def _gather(table, idx_flat):
    mesh = plsc.VectorSubcoreMesh(core_axis_name="c", subcore_axis_name="s")

    @functools.partial(
        pl.kernel,
        mesh=mesh,
        out_type=jax.ShapeDtypeStruct((_NK, _TW), jnp.float32),
        scratch_types=[
            pltpu.VMEM((_G,), jnp.int32),
            pltpu.VMEM((_G, _TW), jnp.float32),
            pltpu.SemaphoreType.DMA,
        ],
    )
    def gk(t_hbm, idx_hbm, o_hbm, idx_v, r1, sem):
        wid = lax.axis_index("s") * 2 + lax.axis_index("c")
        base = wid * _BPW

        def body(i, c):
            off = base + i * _G
            pltpu.sync_copy(idx_hbm.at[pl.ds(off, _G)], idx_v)
            pltpu.async_copy(t_hbm.at[idx_v], r1, sem).wait()
            pltpu.sync_copy(r1, o_hbm.at[pl.ds(off, _G)])
            return c

        lax.fori_loop(0, _BPW // _G, body, 0)

    return gk(table, idx_flat)


_FB = 512
_NP = _FB // _K


def _expand_mat():
    r = lax.broadcasted_iota(jnp.int32, (_FB, _NP), 0) // _K
    c = lax.broadcasted_iota(jnp.int32, (_FB, _NP), 1)
    return (r == c).astype(jnp.float32)


def _leaky(x):
    return jnp.where(x >= 0, x, 0.01 * x)


_FBS = 2048
_NPS = _FBS // _K


def _expand_mat_s():
    r = lax.broadcasted_iota(jnp.int32, (_FBS, _NPS), 0) // _K
    c = lax.broadcasted_iota(jnp.int32, (_FBS, _NPS), 1)
    return (r == c).astype(jnp.float32)


def _pe1stat_body(xg_ref, xl8_ref, pw1t_ref, pb1_ref, st_ref):
    e = _expand_mat_s()
    delta = xg_ref[:, 0:8] - jnp.dot(e, xl8_ref[...], preferred_element_type=jnp.float32)
    pe1 = jnp.dot(delta, pw1t_ref[...], preferred_element_type=jnp.float32) + pb1_ref[...]
    st_ref[0:1, 0:1, 0:_PH] = jnp.sum(pe1, axis=0, keepdims=True).reshape(1, 1, _PH)
    st_ref[0:1, 0:1, _PH:2 * _PH] = jnp.sum(pe1 * pe1, axis=0, keepdims=True).reshape(1, 1, _PH)


def _pe1stat(g, xl8, pw1t8, pb1):
    grid = (_NK // _FBS,)
    return pl.pallas_call(
        _pe1stat_body,
        grid=grid,
        in_specs=[
            pl.BlockSpec((_FBS, 128), lambda b: (b, 4)),
            pl.BlockSpec((_NPS, 8), lambda b: (b, 0)),
            pl.BlockSpec((8, _PH), lambda b: (0, 0)),
            pl.BlockSpec((1, _PH), lambda b: (0, 0)),
        ],
        out_specs=pl.BlockSpec((1, 1, 2 * _PH), lambda b: (b, 0, 0)),
        out_shape=jax.ShapeDtypeStruct((_NK // _FBS, 1, 2 * _PH), jnp.float32),
    )(g, xl8, pw1t8, pb1)


def _passB_body(xg_ref, kg_ref, vg_ref, ql_ref, xl8_ref, pw1t_ref, pb1_ref,
                a1_ref, c1_ref, pw2t_ref, pb2_ref,
                wpre_ref, vout_ref, st_ref):
    e = _expand_mat()
    delta = xg_ref[:, 0:8] - jnp.dot(e, xl8_ref[...], preferred_element_type=jnp.float32)
    pe1 = jnp.dot(delta, pw1t_ref[...], preferred_element_type=jnp.float32) + pb1_ref[...]
    pe1 = _leaky(pe1 * a1_ref[...] + c1_ref[...])
    pe2 = jnp.dot(pe1.astype(jnp.bfloat16), pw2t_ref[...].astype(jnp.bfloat16),
                  preferred_element_type=jnp.float32) + pb2_ref[...]
    qlr = jnp.dot(e, ql_ref[...], preferred_element_type=jnp.float32)
    wpre = qlr - kg_ref[...] + pe2
    wpre_ref[...] = wpre
    vout_ref[...] = vg_ref[...] + pe2
    st_ref[0:1, 0:1, 0:_C] = jnp.sum(wpre, axis=0, keepdims=True).reshape(1, 1, _C)
    st_ref[0:1, 0:1, _C:2 * _C] = jnp.sum(wpre * wpre, axis=0, keepdims=True).reshape(1, 1, _C)


def _passB(g, ql, xl8, pw1t8, pb1, a1, c1, pw2t, pb2):
    grid = (_NK // _FB,)
    return pl.pallas_call(
        _passB_body,
        grid=grid,
        in_specs=[
            pl.BlockSpec((_FB, 128), lambda b: (b, 4)),
            pl.BlockSpec((_FB, _C), lambda b: (b, 0)),
            pl.BlockSpec((_FB, _C), lambda b: (b, 1)),
            pl.BlockSpec((_NP, _C), lambda b: (b, 0)),
            pl.BlockSpec((_NP, 8), lambda b: (b, 0)),
            pl.BlockSpec((8, _PH), lambda b: (0, 0)),
            pl.BlockSpec((1, _PH), lambda b: (0, 0)),
            pl.BlockSpec((1, _PH), lambda b: (0, 0)),
            pl.BlockSpec((1, _PH), lambda b: (0, 0)),
            pl.BlockSpec((_PH, _C), lambda b: (0, 0)),
            pl.BlockSpec((1, _C), lambda b: (0, 0)),
        ],
        out_specs=[
            pl.BlockSpec((_FB, _C), lambda b: (b, 0)),
            pl.BlockSpec((_FB, _C), lambda b: (b, 0)),
            pl.BlockSpec((1, 1, 2 * _C), lambda b: (b, 0, 0)),
        ],
        out_shape=[
            jax.ShapeDtypeStruct((_NK, _C), jnp.float32),
            jax.ShapeDtypeStruct((_NK, _C), jnp.float32),
            jax.ShapeDtypeStruct((_NK // _FB, 1, 2 * _C), jnp.float32),
        ],
    )(g, g, g, ql, xl8, pw1t8, pb1, a1, c1, pw2t, pb2)


def _passC_body(wpre_ref, ag_ref, cg_ref, wwt_ref, wb_ref, h_ref, st_ref):
    s = _leaky(wpre_ref[...] * ag_ref[...] + cg_ref[...])
    h = jnp.dot(s.astype(jnp.bfloat16), wwt_ref[...].astype(jnp.bfloat16),
                preferred_element_type=jnp.float32) + wb_ref[...]
    h_ref[...] = h
    st_ref[0:1, 0:1, 0:_C] = jnp.sum(h, axis=0, keepdims=True).reshape(1, 1, _C)
    st_ref[0:1, 0:1, _C:2 * _C] = jnp.sum(h * h, axis=0, keepdims=True).reshape(1, 1, _C)


def _passC(wpre, ag1, cg1, wwt, wb):
    grid = (_NK // _FB,)
    return pl.pallas_call(
        _passC_body,
        grid=grid,
        in_specs=[
            pl.BlockSpec((_FB, _C), lambda b: (b, 0)),
            pl.BlockSpec((1, _C), lambda b: (0, 0)),
            pl.BlockSpec((1, _C), lambda b: (0, 0)),
            pl.BlockSpec((_C, _C), lambda b: (0, 0)),
            pl.BlockSpec((1, _C), lambda b: (0, 0)),
        ],
        out_specs=[
            pl.BlockSpec((_FB, _C), lambda b: (b, 0)),
            pl.BlockSpec((1, 1, 2 * _C), lambda b: (b, 0, 0)),
        ],
        out_shape=[
            jax.ShapeDtypeStruct((_NK, _C), jnp.float32),
            jax.ShapeDtypeStruct((_NK // _FB, 1, 2 * _C), jnp.float32),
        ],
    )(wpre, ag1, cg1, wwt, wb)


def _tree_red(x, op):
    w = _K
    while w > 1:
        h = w // 2
        x = op(x[:, 0:h], x[:, h:w])
        w = h
    return x


def _passD_body(h_ref, v_ref, ag_ref, cg_ref, out_ref):
    s = _leaky(h_ref[...] * ag_ref[...] + cg_ref[...])
    m = _tree_red(s, jnp.maximum)
    ex = jnp.exp(s - m)
    den = _tree_red(ex, jnp.add)
    w = ex / den
    out_ref[...] = _tree_red(w * v_ref[...], jnp.add)


def _passD(h3, v3, ag2, cg2):
    grid = (_N // _NP,)
    return pl.pallas_call(
        _passD_body,
        grid=grid,
        in_specs=[
            pl.BlockSpec((_NP, _K, _C), lambda b: (b, 0, 0)),
            pl.BlockSpec((_NP, _K, _C), lambda b: (b, 0, 0)),
            pl.BlockSpec((1, 1, _C), lambda b: (0, 0, 0)),
            pl.BlockSpec((1, 1, _C), lambda b: (0, 0, 0)),
        ],
        out_specs=pl.BlockSpec((_NP, 1, _C), lambda b: (b, 0, 0)),
        out_shape=jax.ShapeDtypeStruct((_N, 1, _C), jnp.float32),
    )(h3, v3, ag2, cg2)


def _bn_ab(stats3, width, gamma, beta):
    stats = stats3.reshape(-1, 2 * width)
    s1 = jnp.sum(stats[:, 0:width], axis=0)
    s2 = jnp.sum(stats[:, width:2 * width], axis=0)
    mean = s1 / _NK
    var = s2 / _NK - mean * mean
    a = gamma / jnp.sqrt(var + _EPS)
    b = beta - mean * a
    return a.reshape(1, width), b.reshape(1, width)


def kernel(fea_i, fea_last, xyz_i, xyz_last, batch, t_i,
           p_w1, p_b1, p_g1, p_be1, p_w2, p_b2,
           q_w, q_b, k_w, k_b, v_w, v_b,
           w_g1, w_be1, w_w, w_b, w_g2, w_be2):
    f32 = jnp.float32
    t_i = jnp.asarray(t_i, f32)

    ql, kv = _proj(fea_last, fea_i,
                   q_w.T, q_b.reshape(1, _C), k_w.T, k_b.reshape(1, _C),
                   v_w.T, v_b.reshape(1, _C))

    by = batch.reshape(_N, 1)
    bx = batch.reshape(1, _N)
    y8 = jnp.concatenate([xyz_last, jnp.zeros((_N, 5), f32)], axis=1)
    xt8 = jnp.concatenate([xyz_i.T, jnp.zeros((5, _N), f32)], axis=0)
    bounds = jnp.searchsorted(batch, jnp.arange(9, dtype=jnp.int32),
                              side="left").astype(jnp.int32)
    bf = batch[0::_RB]
    bl = batch[_RB - 1::_RB]
    tlo = (bounds[bf] // _CT).astype(jnp.int32)
    thi = ((bounds[bl + 1] + _CT - 1) // _CT).astype(jnp.int32)
    idx = _knn(y8, by, xt8, bx, tlo, thi)
    idx_flat = idx.reshape(_NK)

    table = jnp.concatenate(
        [kv, xyz_i, jnp.full((_N, 1), t_i, f32), jnp.zeros((_N, 124), f32)], axis=1)
    g = _gather(table, idx_flat)

    xl8 = jnp.concatenate(
        [xyz_last, jnp.full((_N, 1), _T_LAST, f32), jnp.zeros((_N, 4), f32)], axis=1)
    pw1t8 = jnp.concatenate([p_w1.T, jnp.zeros((4, _PH), f32)], axis=0)
    pb1 = p_b1.reshape(1, _PH)

    st1 = _pe1stat(g, xl8, pw1t8, pb1)
    a1, c1 = _bn_ab(st1, _PH, p_g1, p_be1)

    wpre, vout, st2 = _passB(g, ql, xl8, pw1t8, pb1, a1, c1,
                             p_w2.T, p_b2.reshape(1, _C))
    ag1, cg1 = _bn_ab(st2, _C, w_g1, w_be1)

    h, st3 = _passC(wpre, ag1, cg1, w_w.T, w_b.reshape(1, _C))
    ag2, cg2 = _bn_ab(st3, _C, w_g2, w_be2)

    out3 = _passD(h.reshape(_N, _K, _C), vout.reshape(_N, _K, _C),
                  ag2.reshape(1, 1, _C), cg2.reshape(1, 1, _C))
    return out3.reshape(_N, _C)


# speedup vs baseline: 6.6596x; 1.3443x over previous
import functools

import jax
import jax.numpy as jnp
from jax import lax
from jax.experimental import pallas as pl
from jax.experimental.pallas import tpu as pltpu
from jax.experimental.pallas import tpu_sc as plsc

_N = 8192
_K = 16
_C = 256
_PH = 64
_T_LAST = 1.0
_EPS = 1e-5
_NK = _N * _K

_MASKVAL = 1e38

_RBP = 256


def _proj_body(fl_ref, fi_ref, qwt_ref, qb_ref, kwt_ref, kb_ref, vwt_ref, vb_ref,
               ql_ref, kv_ref):
    fl = fl_ref[...].astype(jnp.bfloat16)
    fi = fi_ref[...].astype(jnp.bfloat16)
    ql_ref[...] = jnp.dot(fl, qwt_ref[...].astype(jnp.bfloat16),
                          preferred_element_type=jnp.float32) + qb_ref[...]
    kv_ref[:, 0:_C] = jnp.dot(fi, kwt_ref[...].astype(jnp.bfloat16),
                              preferred_element_type=jnp.float32) + kb_ref[...]
    kv_ref[:, _C:2 * _C] = jnp.dot(fi, vwt_ref[...].astype(jnp.bfloat16),
                                   preferred_element_type=jnp.float32) + vb_ref[...]


def _proj(fea_last, fea_i, q_wt, q_b, k_wt, k_b, v_wt, v_b):
    grid = (_N // _RBP,)
    return pl.pallas_call(
        _proj_body,
        grid=grid,
        in_specs=[
            pl.BlockSpec((_RBP, _C), lambda b: (b, 0)),
            pl.BlockSpec((_RBP, _C), lambda b: (b, 0)),
            pl.BlockSpec((_C, _C), lambda b: (0, 0)),
            pl.BlockSpec((1, _C), lambda b: (0, 0)),
            pl.BlockSpec((_C, _C), lambda b: (0, 0)),
            pl.BlockSpec((1, _C), lambda b: (0, 0)),
            pl.BlockSpec((_C, _C), lambda b: (0, 0)),
            pl.BlockSpec((1, _C), lambda b: (0, 0)),
        ],
        out_specs=[
            pl.BlockSpec((_RBP, _C), lambda b: (b, 0)),
            pl.BlockSpec((_RBP, 2 * _C), lambda b: (b, 0)),
        ],
        out_shape=[
            jax.ShapeDtypeStruct((_N, _C), jnp.float32),
            jax.ShapeDtypeStruct((_N, 2 * _C), jnp.float32),
        ],
    )(fea_last, fea_i, q_wt, q_b, k_wt, k_b, v_wt, v_b)


_RB = 128
_CT = 512
_NT = _N // _CT


def _knn_body(tlo_ref, thi_ref, yt_ref, by_ref, xr_ref, bx_ref, idx_ref, d_ref):
    pid = pl.program_id(0)
    tlo = tlo_ref[pid]
    thi = thi_ref[pid]
    yt = yt_ref[...]
    ybt = yt.astype(jnp.bfloat16)
    by = by_ref[...]
    big = jnp.int32(2 ** 30)
    inf = jnp.float32(jnp.inf)
    yy = yt[0:1, :] * yt[0:1, :] + yt[1:2, :] * yt[1:2, :] + yt[2:3, :] * yt[2:3, :]

    am = jnp.full((1, _RB), -1, jnp.int32)
    for k in range(_K):
        am_prev = am

        def step(t, carry, am_prev=am_prev, first=(k == 0), last=(k == _K - 1)):
            m, am = carry
            sl = pl.ds(t * _CT, _CT)
            rowi = lax.broadcasted_iota(jnp.int32, (_CT, _RB), 0) + t * _CT
            if first:
                xs = xr_ref[sl, :]
                xx = (xs[:, 0:1] * xs[:, 0:1] + xs[:, 1:2] * xs[:, 1:2]
                      + xs[:, 2:3] * xs[:, 2:3])
                dot = jnp.dot(xs.astype(jnp.bfloat16), ybt,
                              preferred_element_type=jnp.float32)
                tile = yy + xx - 2.0 * dot
                bx = bx_ref[sl, :]
                tile = jnp.where(bx != by, jnp.float32(_MASKVAL), tile)
            else:
                tile = jnp.where(rowi == am_prev, inf, d_ref[sl, :])
            if not last:
                d_ref[sl, :] = tile
            tmin = jnp.min(tile, axis=0, keepdims=True)
            tam = jnp.min(jnp.where(tile <= tmin, rowi, big), axis=0, keepdims=True)
            upd = tmin < m
            return (jnp.where(upd, tmin, m), jnp.where(upd, tam, am))

        m, am = lax.fori_loop(
            tlo, thi, step,
            (jnp.full((1, _RB), inf, jnp.float32), jnp.full((1, _RB), big, jnp.int32)))
        idx_ref[k:k + 1, :] = am


def _knn(yt8, byr, x8r, bxc, tlo, thi):
    grid_spec = pltpu.PrefetchScalarGridSpec(
        num_scalar_prefetch=2,
        grid=(_N // _RB,),
        in_specs=[
            pl.BlockSpec((8, _RB), lambda b, *_: (0, b)),
            pl.BlockSpec((1, _RB), lambda b, *_: (0, b)),
            pl.BlockSpec((_N, 8), lambda b, *_: (0, 0)),
            pl.BlockSpec((_N, 1), lambda b, *_: (0, 0)),
        ],
        out_specs=pl.BlockSpec((_K, _RB), lambda b, *_: (0, b)),
        scratch_shapes=[pltpu.VMEM((_N, _RB), jnp.float32)],
    )
    return pl.pallas_call(
        _knn_body,
        grid_spec=grid_spec,
        out_shape=jax.ShapeDtypeStruct((_K, _N), jnp.int32),
    )(tlo, thi, yt8, byr, x8r, bxc)


_NW = 32
_BPW = _NK // _NW
_G = 128


_TW = 2 * _C + 128


def _gather(table, idx_flat):
    mesh = plsc.VectorSubcoreMesh(core_axis_name="c", subcore_axis_name="s")

    @functools.partial(
        pl.kernel,
        mesh=mesh,
        out_type=jax.ShapeDtypeStruct((_NK, _TW), jnp.float32),
        scratch_types=[
            pltpu.VMEM((_G,), jnp.int32),
            pltpu.VMEM((_G, _TW), jnp.float32),
            pltpu.SemaphoreType.DMA,
        ],
    )
    def gk(t_hbm, idx_hbm, o_hbm, idx_v, r1, sem):
        wid = lax.axis_index("s") * 2 + lax.axis_index("c")
        base = wid * _BPW

        def body(i, c):
            off = base + i * _G
            pltpu.sync_copy(idx_hbm.at[pl.ds(off, _G)], idx_v)
            pltpu.async_copy(t_hbm.at[idx_v], r1, sem).wait()
            pltpu.sync_copy(r1, o_hbm.at[pl.ds(off, _G)])
            return c

        lax.fori_loop(0, _BPW // _G, body, 0)

    return gk(table, idx_flat)


_FB = 512
_NP = _FB // _K


def _expand_mat():
    r = lax.broadcasted_iota(jnp.int32, (_FB, _NP), 0) // _K
    c = lax.broadcasted_iota(jnp.int32, (_FB, _NP), 1)
    return (r == c).astype(jnp.float32)


def _leaky(x):
    return jnp.where(x >= 0, x, 0.01 * x)


_FBS = 2048
_NPS = _FBS // _K


def _expand_mat_s():
    r = lax.broadcasted_iota(jnp.int32, (_FBS, _NPS), 0) // _K
    c = lax.broadcasted_iota(jnp.int32, (_FBS, _NPS), 1)
    return (r == c).astype(jnp.float32)


def _pe1stat_body(xg_ref, xl8_ref, pw1t_ref, pb1_ref, st_ref):
    e = _expand_mat_s()
    delta = xg_ref[:, 0:8] - jnp.dot(e, xl8_ref[...], preferred_element_type=jnp.float32)
    pe1 = jnp.dot(delta, pw1t_ref[...], preferred_element_type=jnp.float32) + pb1_ref[...]
    st_ref[0:1, 0:1, 0:_PH] = jnp.sum(pe1, axis=0, keepdims=True).reshape(1, 1, _PH)
    st_ref[0:1, 0:1, _PH:2 * _PH] = jnp.sum(pe1 * pe1, axis=0, keepdims=True).reshape(1, 1, _PH)


def _pe1stat(g, xl8, pw1t8, pb1):
    grid = (_NK // _FBS,)
    return pl.pallas_call(
        _pe1stat_body,
        grid=grid,
        in_specs=[
            pl.BlockSpec((_FBS, 128), lambda b: (b, 4)),
            pl.BlockSpec((_NPS, 8), lambda b: (b, 0)),
            pl.BlockSpec((8, _PH), lambda b: (0, 0)),
            pl.BlockSpec((1, _PH), lambda b: (0, 0)),
        ],
        out_specs=pl.BlockSpec((1, 1, 2 * _PH), lambda b: (b, 0, 0)),
        out_shape=jax.ShapeDtypeStruct((_NK // _FBS, 1, 2 * _PH), jnp.float32),
    )(g, xl8, pw1t8, pb1)


def _passB_body(xg_ref, kg_ref, vg_ref, ql_ref, xl8_ref, pw1t_ref, pb1_ref,
                a1_ref, c1_ref, pw2t_ref, pb2_ref,
                wpre_ref, vout_ref, st_ref):
    e = _expand_mat()
    delta = xg_ref[:, 0:8] - jnp.dot(e, xl8_ref[...], preferred_element_type=jnp.float32)
    pe1 = jnp.dot(delta, pw1t_ref[...], preferred_element_type=jnp.float32) + pb1_ref[...]
    pe1 = _leaky(pe1 * a1_ref[...] + c1_ref[...])
    pe2 = jnp.dot(pe1.astype(jnp.bfloat16), pw2t_ref[...].astype(jnp.bfloat16),
                  preferred_element_type=jnp.float32) + pb2_ref[...]
    qlr = jnp.dot(e, ql_ref[...], preferred_element_type=jnp.float32)
    wpre = qlr - kg_ref[...] + pe2
    wpre_ref[...] = wpre
    vout_ref[...] = vg_ref[...] + pe2
    st_ref[0:1, 0:1, 0:_C] = jnp.sum(wpre, axis=0, keepdims=True).reshape(1, 1, _C)
    st_ref[0:1, 0:1, _C:2 * _C] = jnp.sum(wpre * wpre, axis=0, keepdims=True).reshape(1, 1, _C)


def _passB(g, ql, xl8, pw1t8, pb1, a1, c1, pw2t, pb2):
    grid = (_NK // _FB,)
    return pl.pallas_call(
        _passB_body,
        grid=grid,
        in_specs=[
            pl.BlockSpec((_FB, 128), lambda b: (b, 4)),
            pl.BlockSpec((_FB, _C), lambda b: (b, 0)),
            pl.BlockSpec((_FB, _C), lambda b: (b, 1)),
            pl.BlockSpec((_NP, _C), lambda b: (b, 0)),
            pl.BlockSpec((_NP, 8), lambda b: (b, 0)),
            pl.BlockSpec((8, _PH), lambda b: (0, 0)),
            pl.BlockSpec((1, _PH), lambda b: (0, 0)),
            pl.BlockSpec((1, _PH), lambda b: (0, 0)),
            pl.BlockSpec((1, _PH), lambda b: (0, 0)),
            pl.BlockSpec((_PH, _C), lambda b: (0, 0)),
            pl.BlockSpec((1, _C), lambda b: (0, 0)),
        ],
        out_specs=[
            pl.BlockSpec((_FB, _C), lambda b: (b, 0)),
            pl.BlockSpec((_FB, _C), lambda b: (b, 0)),
            pl.BlockSpec((1, 1, 2 * _C), lambda b: (b, 0, 0)),
        ],
        out_shape=[
            jax.ShapeDtypeStruct((_NK, _C), jnp.float32),
            jax.ShapeDtypeStruct((_NK, _C), jnp.float32),
            jax.ShapeDtypeStruct((_NK // _FB, 1, 2 * _C), jnp.float32),
        ],
    )(g, g, g, ql, xl8, pw1t8, pb1, a1, c1, pw2t, pb2)


def _passC_body(wpre_ref, ag_ref, cg_ref, wwt_ref, wb_ref, h_ref, st_ref):
    s = _leaky(wpre_ref[...] * ag_ref[...] + cg_ref[...])
    h = jnp.dot(s.astype(jnp.bfloat16), wwt_ref[...].astype(jnp.bfloat16),
                preferred_element_type=jnp.float32) + wb_ref[...]
    h_ref[...] = h
    st_ref[0:1, 0:1, 0:_C] = jnp.sum(h, axis=0, keepdims=True).reshape(1, 1, _C)
    st_ref[0:1, 0:1, _C:2 * _C] = jnp.sum(h * h, axis=0, keepdims=True).reshape(1, 1, _C)


def _passC(wpre, ag1, cg1, wwt, wb):
    grid = (_NK // _FB,)
    return pl.pallas_call(
        _passC_body,
        grid=grid,
        in_specs=[
            pl.BlockSpec((_FB, _C), lambda b: (b, 0)),
            pl.BlockSpec((1, _C), lambda b: (0, 0)),
            pl.BlockSpec((1, _C), lambda b: (0, 0)),
            pl.BlockSpec((_C, _C), lambda b: (0, 0)),
            pl.BlockSpec((1, _C), lambda b: (0, 0)),
        ],
        out_specs=[
            pl.BlockSpec((_FB, _C), lambda b: (b, 0)),
            pl.BlockSpec((1, 1, 2 * _C), lambda b: (b, 0, 0)),
        ],
        out_shape=[
            jax.ShapeDtypeStruct((_NK, _C), jnp.float32),
            jax.ShapeDtypeStruct((_NK // _FB, 1, 2 * _C), jnp.float32),
        ],
    )(wpre, ag1, cg1, wwt, wb)


def _tree_red(x, op):
    w = _K
    while w > 1:
        h = w // 2
        x = op(x[:, 0:h], x[:, h:w])
        w = h
    return x


def _passD_body(h_ref, v_ref, ag_ref, cg_ref, out_ref):
    s = _leaky(h_ref[...] * ag_ref[...] + cg_ref[...])
    m = _tree_red(s, jnp.maximum)
    ex = jnp.exp(s - m)
    den = _tree_red(ex, jnp.add)
    w = ex / den
    out_ref[...] = _tree_red(w * v_ref[...], jnp.add)


def _passD(h3, v3, ag2, cg2):
    grid = (_N // _NP,)
    return pl.pallas_call(
        _passD_body,
        grid=grid,
        in_specs=[
            pl.BlockSpec((_NP, _K, _C), lambda b: (b, 0, 0)),
            pl.BlockSpec((_NP, _K, _C), lambda b: (b, 0, 0)),
            pl.BlockSpec((1, 1, _C), lambda b: (0, 0, 0)),
            pl.BlockSpec((1, 1, _C), lambda b: (0, 0, 0)),
        ],
        out_specs=pl.BlockSpec((_NP, 1, _C), lambda b: (b, 0, 0)),
        out_shape=jax.ShapeDtypeStruct((_N, 1, _C), jnp.float32),
    )(h3, v3, ag2, cg2)


def _bn_ab(stats3, width, gamma, beta):
    stats = stats3.reshape(-1, 2 * width)
    s1 = jnp.sum(stats[:, 0:width], axis=0)
    s2 = jnp.sum(stats[:, width:2 * width], axis=0)
    mean = s1 / _NK
    var = s2 / _NK - mean * mean
    a = gamma / jnp.sqrt(var + _EPS)
    b = beta - mean * a
    return a.reshape(1, width), b.reshape(1, width)


def kernel(fea_i, fea_last, xyz_i, xyz_last, batch, t_i,
           p_w1, p_b1, p_g1, p_be1, p_w2, p_b2,
           q_w, q_b, k_w, k_b, v_w, v_b,
           w_g1, w_be1, w_w, w_b, w_g2, w_be2):
    f32 = jnp.float32
    t_i = jnp.asarray(t_i, f32)

    ql, kv = _proj(fea_last, fea_i,
                   q_w.T, q_b.reshape(1, _C), k_w.T, k_b.reshape(1, _C),
                   v_w.T, v_b.reshape(1, _C))

    byr = batch.reshape(1, _N)
    bxc = batch.reshape(_N, 1)
    yt8 = jnp.concatenate([xyz_last.T, jnp.zeros((5, _N), f32)], axis=0)
    x8r = jnp.concatenate([xyz_i, jnp.zeros((_N, 5), f32)], axis=1)
    bounds = jnp.searchsorted(batch, jnp.arange(9, dtype=jnp.int32),
                              side="left").astype(jnp.int32)
    bf = batch[0::_RB]
    bl = batch[_RB - 1::_RB]
    tlo = (bounds[bf] // _CT).astype(jnp.int32)
    thi = ((bounds[bl + 1] + _CT - 1) // _CT).astype(jnp.int32)
    idx = _knn(yt8, byr, x8r, bxc, tlo, thi)
    idx_flat = idx.T.reshape(_NK)

    table = jnp.concatenate(
        [kv, xyz_i, jnp.full((_N, 1), t_i, f32), jnp.zeros((_N, 124), f32)], axis=1)
    g = _gather(table, idx_flat)

    xl8 = jnp.concatenate(
        [xyz_last, jnp.full((_N, 1), _T_LAST, f32), jnp.zeros((_N, 4), f32)], axis=1)
    pw1t8 = jnp.concatenate([p_w1.T, jnp.zeros((4, _PH), f32)], axis=0)
    pb1 = p_b1.reshape(1, _PH)

    st1 = _pe1stat(g, xl8, pw1t8, pb1)
    a1, c1 = _bn_ab(st1, _PH, p_g1, p_be1)

    wpre, vout, st2 = _passB(g, ql, xl8, pw1t8, pb1, a1, c1,
                             p_w2.T, p_b2.reshape(1, _C))
    ag1, cg1 = _bn_ab(st2, _C, w_g1, w_be1)

    h, st3 = _passC(wpre, ag1, cg1, w_w.T, w_b.reshape(1, _C))
    ag2, cg2 = _bn_ab(st3, _C, w_g2, w_be2)

    out3 = _passD(h.reshape(_N, _K, _C), vout.reshape(_N, _K, _C),
                  ag2.reshape(1, 1, _C), cg2.reshape(1, 1, _C))
    return out3.reshape(_N, _C)


# speedup vs baseline: 7.3005x; 1.0962x over previous
import functools

import jax
import jax.numpy as jnp
from jax import lax
from jax.experimental import pallas as pl
from jax.experimental.pallas import tpu as pltpu
from jax.experimental.pallas import tpu_sc as plsc

_N = 8192
_K = 16
_C = 256
_PH = 64
_T_LAST = 1.0
_EPS = 1e-5
_NK = _N * _K

_MASKVAL = 1e38

_RBP = 256


def _proj_body(fl_ref, fi_ref, qwt_ref, qb_ref, kwt_ref, kb_ref, vwt_ref, vb_ref,
               ql_ref, kv_ref):
    fl = fl_ref[...].astype(jnp.bfloat16)
    fi = fi_ref[...].astype(jnp.bfloat16)
    ql_ref[...] = jnp.dot(fl, qwt_ref[...].astype(jnp.bfloat16),
                          preferred_element_type=jnp.float32) + qb_ref[...]
    kv_ref[:, 0:_C] = jnp.dot(fi, kwt_ref[...].astype(jnp.bfloat16),
                              preferred_element_type=jnp.float32) + kb_ref[...]
    kv_ref[:, _C:2 * _C] = jnp.dot(fi, vwt_ref[...].astype(jnp.bfloat16),
                                   preferred_element_type=jnp.float32) + vb_ref[...]


def _proj(fea_last, fea_i, q_wt, q_b, k_wt, k_b, v_wt, v_b):
    grid = (_N // _RBP,)
    return pl.pallas_call(
        _proj_body,
        grid=grid,
        in_specs=[
            pl.BlockSpec((_RBP, _C), lambda b: (b, 0)),
            pl.BlockSpec((_RBP, _C), lambda b: (b, 0)),
            pl.BlockSpec((_C, _C), lambda b: (0, 0)),
            pl.BlockSpec((1, _C), lambda b: (0, 0)),
            pl.BlockSpec((_C, _C), lambda b: (0, 0)),
            pl.BlockSpec((1, _C), lambda b: (0, 0)),
            pl.BlockSpec((_C, _C), lambda b: (0, 0)),
            pl.BlockSpec((1, _C), lambda b: (0, 0)),
        ],
        out_specs=[
            pl.BlockSpec((_RBP, _C), lambda b: (b, 0)),
            pl.BlockSpec((_RBP, 2 * _C), lambda b: (b, 0)),
        ],
        out_shape=[
            jax.ShapeDtypeStruct((_N, _C), jnp.float32),
            jax.ShapeDtypeStruct((_N, 2 * _C), jnp.float32),
        ],
    )(fea_last, fea_i, q_wt, q_b, k_wt, k_b, v_wt, v_b)


_RB = 256
_CT = 512
_NT = _N // _CT


def _knn_body(tlo_ref, thi_ref, yt_ref, by_ref, xr_ref, bx_ref, idx_ref, d_ref):
    pid = pl.program_id(0)
    tlo = tlo_ref[pid]
    thi = thi_ref[pid]
    yt = yt_ref[...]
    ybt = yt.astype(jnp.bfloat16)
    by = by_ref[...]
    big = jnp.int32(2 ** 30)
    inf = jnp.float32(jnp.inf)
    yy = yt[0:1, :] * yt[0:1, :] + yt[1:2, :] * yt[1:2, :] + yt[2:3, :] * yt[2:3, :]

    am = jnp.full((1, _RB), -1, jnp.int32)
    for k in range(_K):
        am_prev = am

        def step(t, carry, am_prev=am_prev, first=(k == 0), last=(k == _K - 1)):
            m, am = carry
            sl = pl.ds(t * _CT, _CT)
            rowi = lax.broadcasted_iota(jnp.int32, (_CT, _RB), 0) + t * _CT
            if first:
                xs = xr_ref[sl, :]
                xx = (xs[:, 0:1] * xs[:, 0:1] + xs[:, 1:2] * xs[:, 1:2]
                      + xs[:, 2:3] * xs[:, 2:3])
                dot = jnp.dot(xs.astype(jnp.bfloat16), ybt,
                              preferred_element_type=jnp.float32)
                tile = yy + xx - 2.0 * dot
                bx = bx_ref[sl, :]
                tile = jnp.where(bx != by, jnp.float32(_MASKVAL), tile)
            else:
                tile = jnp.where(rowi == am_prev, inf, d_ref[sl, :])
            if not last:
                d_ref[sl, :] = tile
            tmin = jnp.min(tile, axis=0, keepdims=True)
            tam = jnp.min(jnp.where(tile <= tmin, rowi, big), axis=0, keepdims=True)
            upd = tmin < m
            return (jnp.where(upd, tmin, m), jnp.where(upd, tam, am))

        m, am = lax.fori_loop(
            tlo, thi, step,
            (jnp.full((1, _RB), inf, jnp.float32), jnp.full((1, _RB), big, jnp.int32)))
        idx_ref[k:k + 1, :] = am


def _knn(yt8, byr, x8r, bxc, tlo, thi):
    grid_spec = pltpu.PrefetchScalarGridSpec(
        num_scalar_prefetch=2,
        grid=(_N // _RB,),
        in_specs=[
            pl.BlockSpec((8, _RB), lambda b, *_: (0, b)),
            pl.BlockSpec((1, _RB), lambda b, *_: (0, b)),
            pl.BlockSpec((_N, 8), lambda b, *_: (0, 0)),
            pl.BlockSpec((_N, 1), lambda b, *_: (0, 0)),
        ],
        out_specs=pl.BlockSpec((_K, _RB), lambda b, *_: (0, b)),
        scratch_shapes=[pltpu.VMEM((_N, _RB), jnp.float32)],
    )
    return pl.pallas_call(
        _knn_body,
        grid_spec=grid_spec,
        out_shape=jax.ShapeDtypeStruct((_K, _N), jnp.int32),
    )(tlo, thi, yt8, byr, x8r, bxc)


_NW = 32
_BPW = _NK // _NW
_G = 128


_TW = 2 * _C + 128


def _gather(table, idx_flat):
    mesh = plsc.VectorSubcoreMesh(core_axis_name="c", subcore_axis_name="s")

    @functools.partial(
        pl.kernel,
        mesh=mesh,
        out_type=jax.ShapeDtypeStruct((_NK, _TW), jnp.float32),
        scratch_types=[
            pltpu.VMEM((_G,), jnp.int32),
            pltpu.VMEM((_G, _TW), jnp.float32),
            pltpu.SemaphoreType.DMA,
        ],
    )
    def gk(t_hbm, idx_hbm, o_hbm, idx_v, r1, sem):
        wid = lax.axis_index("s") * 2 + lax.axis_index("c")
        base = wid * _BPW

        def body(i, c):
            off = base + i * _G
            pltpu.sync_copy(idx_hbm.at[pl.ds(off, _G)], idx_v)
            pltpu.async_copy(t_hbm.at[idx_v], r1, sem).wait()
            pltpu.sync_copy(r1, o_hbm.at[pl.ds(off, _G)])
            return c

        lax.fori_loop(0, _BPW // _G, body, 0)

    return gk(table, idx_flat)


_FB = 512
_NP = _FB // _K


def _expand_mat():
    r = lax.broadcasted_iota(jnp.int32, (_FB, _NP), 0) // _K
    c = lax.broadcasted_iota(jnp.int32, (_FB, _NP), 1)
    return (r == c).astype(jnp.float32)


def _leaky(x):
    return jnp.where(x >= 0, x, 0.01 * x)


_FBS = 2048
_NPS = _FBS // _K


def _expand_mat_s():
    r = lax.broadcasted_iota(jnp.int32, (_FBS, _NPS), 0) // _K
    c = lax.broadcasted_iota(jnp.int32, (_FBS, _NPS), 1)
    return (r == c).astype(jnp.float32)


def _pe1stat_body(xg_ref, xl8_ref, pw1t_ref, pb1_ref, st_ref):
    e = _expand_mat_s()
    delta = xg_ref[:, 0:8] - jnp.dot(e, xl8_ref[...], preferred_element_type=jnp.float32)
    pe1 = jnp.dot(delta, pw1t_ref[...], preferred_element_type=jnp.float32) + pb1_ref[...]
    st_ref[0:1, 0:1, 0:_PH] = jnp.sum(pe1, axis=0, keepdims=True).reshape(1, 1, _PH)
    st_ref[0:1, 0:1, _PH:2 * _PH] = jnp.sum(pe1 * pe1, axis=0, keepdims=True).reshape(1, 1, _PH)


def _pe1stat(g, xl8, pw1t8, pb1):
    grid = (_NK // _FBS,)
    return pl.pallas_call(
        _pe1stat_body,
        grid=grid,
        in_specs=[
            pl.BlockSpec((_FBS, 128), lambda b: (b, 4)),
            pl.BlockSpec((_NPS, 8), lambda b: (b, 0)),
            pl.BlockSpec((8, _PH), lambda b: (0, 0)),
            pl.BlockSpec((1, _PH), lambda b: (0, 0)),
        ],
        out_specs=pl.BlockSpec((1, 1, 2 * _PH), lambda b: (b, 0, 0)),
        out_shape=jax.ShapeDtypeStruct((_NK // _FBS, 1, 2 * _PH), jnp.float32),
    )(g, xl8, pw1t8, pb1)


def _passB_body(xg_ref, kg_ref, vg_ref, ql_ref, xl8_ref, pw1t_ref, pb1_ref,
                a1_ref, c1_ref, pw2t_ref, pb2_ref,
                wpre_ref, vout_ref, st_ref):
    e = _expand_mat()
    delta = xg_ref[:, 0:8] - jnp.dot(e, xl8_ref[...], preferred_element_type=jnp.float32)
    pe1 = jnp.dot(delta, pw1t_ref[...], preferred_element_type=jnp.float32) + pb1_ref[...]
    pe1 = _leaky(pe1 * a1_ref[...] + c1_ref[...])
    pe2 = jnp.dot(pe1.astype(jnp.bfloat16), pw2t_ref[...].astype(jnp.bfloat16),
                  preferred_element_type=jnp.float32) + pb2_ref[...]
    qlr = jnp.dot(e, ql_ref[...], preferred_element_type=jnp.float32)
    wpre = qlr - kg_ref[...] + pe2
    wpre_ref[...] = wpre
    vout_ref[...] = vg_ref[...] + pe2
    st_ref[0:1, 0:1, 0:_C] = jnp.sum(wpre, axis=0, keepdims=True).reshape(1, 1, _C)
    st_ref[0:1, 0:1, _C:2 * _C] = jnp.sum(wpre * wpre, axis=0, keepdims=True).reshape(1, 1, _C)


def _passB(g, ql, xl8, pw1t8, pb1, a1, c1, pw2t, pb2):
    grid = (_NK // _FB,)
    return pl.pallas_call(
        _passB_body,
        grid=grid,
        in_specs=[
            pl.BlockSpec((_FB, 128), lambda b: (b, 4)),
            pl.BlockSpec((_FB, _C), lambda b: (b, 0)),
            pl.BlockSpec((_FB, _C), lambda b: (b, 1)),
            pl.BlockSpec((_NP, _C), lambda b: (b, 0)),
            pl.BlockSpec((_NP, 8), lambda b: (b, 0)),
            pl.BlockSpec((8, _PH), lambda b: (0, 0)),
            pl.BlockSpec((1, _PH), lambda b: (0, 0)),
            pl.BlockSpec((1, _PH), lambda b: (0, 0)),
            pl.BlockSpec((1, _PH), lambda b: (0, 0)),
            pl.BlockSpec((_PH, _C), lambda b: (0, 0)),
            pl.BlockSpec((1, _C), lambda b: (0, 0)),
        ],
        out_specs=[
            pl.BlockSpec((_FB, _C), lambda b: (b, 0)),
            pl.BlockSpec((_FB, _C), lambda b: (b, 0)),
            pl.BlockSpec((1, 1, 2 * _C), lambda b: (b, 0, 0)),
        ],
        out_shape=[
            jax.ShapeDtypeStruct((_NK, _C), jnp.float32),
            jax.ShapeDtypeStruct((_NK, _C), jnp.float32),
            jax.ShapeDtypeStruct((_NK // _FB, 1, 2 * _C), jnp.float32),
        ],
    )(g, g, g, ql, xl8, pw1t8, pb1, a1, c1, pw2t, pb2)


def _passC_body(wpre_ref, ag_ref, cg_ref, wwt_ref, wb_ref, h_ref, st_ref):
    s = _leaky(wpre_ref[...] * ag_ref[...] + cg_ref[...])
    h = jnp.dot(s.astype(jnp.bfloat16), wwt_ref[...].astype(jnp.bfloat16),
                preferred_element_type=jnp.float32) + wb_ref[...]
    h_ref[...] = h
    st_ref[0:1, 0:1, 0:_C] = jnp.sum(h, axis=0, keepdims=True).reshape(1, 1, _C)
    st_ref[0:1, 0:1, _C:2 * _C] = jnp.sum(h * h, axis=0, keepdims=True).reshape(1, 1, _C)


def _passC(wpre, ag1, cg1, wwt, wb):
    grid = (_NK // _FB,)
    return pl.pallas_call(
        _passC_body,
        grid=grid,
        in_specs=[
            pl.BlockSpec((_FB, _C), lambda b: (b, 0)),
            pl.BlockSpec((1, _C), lambda b: (0, 0)),
            pl.BlockSpec((1, _C), lambda b: (0, 0)),
            pl.BlockSpec((_C, _C), lambda b: (0, 0)),
            pl.BlockSpec((1, _C), lambda b: (0, 0)),
        ],
        out_specs=[
            pl.BlockSpec((_FB, _C), lambda b: (b, 0)),
            pl.BlockSpec((1, 1, 2 * _C), lambda b: (b, 0, 0)),
        ],
        out_shape=[
            jax.ShapeDtypeStruct((_NK, _C), jnp.float32),
            jax.ShapeDtypeStruct((_NK // _FB, 1, 2 * _C), jnp.float32),
        ],
    )(wpre, ag1, cg1, wwt, wb)


def _tree_red(x, op):
    w = _K
    while w > 1:
        h = w // 2
        x = op(x[:, 0:h], x[:, h:w])
        w = h
    return x


def _passD_body(h_ref, v_ref, ag_ref, cg_ref, out_ref):
    s = _leaky(h_ref[...] * ag_ref[...] + cg_ref[...])
    m = _tree_red(s, jnp.maximum)
    ex = jnp.exp(s - m)
    den = _tree_red(ex, jnp.add)
    w = ex / den
    out_ref[...] = _tree_red(w * v_ref[...], jnp.add)


_NPD = 64


def _passD(h3, v3, ag2, cg2):
    grid = (_N // _NPD,)
    return pl.pallas_call(
        _passD_body,
        grid=grid,
        in_specs=[
            pl.BlockSpec((_NPD, _K, _C), lambda b: (b, 0, 0)),
            pl.BlockSpec((_NPD, _K, _C), lambda b: (b, 0, 0)),
            pl.BlockSpec((1, 1, _C), lambda b: (0, 0, 0)),
            pl.BlockSpec((1, 1, _C), lambda b: (0, 0, 0)),
        ],
        out_specs=pl.BlockSpec((_NPD, 1, _C), lambda b: (b, 0, 0)),
        out_shape=jax.ShapeDtypeStruct((_N, 1, _C), jnp.float32),
    )(h3, v3, ag2, cg2)


def _bn_ab(stats3, width, gamma, beta):
    stats = stats3.reshape(-1, 2 * width)
    s1 = jnp.sum(stats[:, 0:width], axis=0)
    s2 = jnp.sum(stats[:, width:2 * width], axis=0)
    mean = s1 / _NK
    var = s2 / _NK - mean * mean
    a = gamma / jnp.sqrt(var + _EPS)
    b = beta - mean * a
    return a.reshape(1, width), b.reshape(1, width)


def kernel(fea_i, fea_last, xyz_i, xyz_last, batch, t_i,
           p_w1, p_b1, p_g1, p_be1, p_w2, p_b2,
           q_w, q_b, k_w, k_b, v_w, v_b,
           w_g1, w_be1, w_w, w_b, w_g2, w_be2):
    f32 = jnp.float32
    t_i = jnp.asarray(t_i, f32)

    ql, kv = _proj(fea_last, fea_i,
                   q_w.T, q_b.reshape(1, _C), k_w.T, k_b.reshape(1, _C),
                   v_w.T, v_b.reshape(1, _C))

    byr = batch.reshape(1, _N)
    bxc = batch.reshape(_N, 1)
    yt8 = jnp.concatenate([xyz_last.T, jnp.zeros((5, _N), f32)], axis=0)
    x8r = jnp.concatenate([xyz_i, jnp.zeros((_N, 5), f32)], axis=1)
    bounds = jnp.searchsorted(batch, jnp.arange(9, dtype=jnp.int32),
                              side="left").astype(jnp.int32)
    bf = batch[0::_RB]
    bl = batch[_RB - 1::_RB]
    tlo = (bounds[bf] // _CT).astype(jnp.int32)
    thi = ((bounds[bl + 1] + _CT - 1) // _CT).astype(jnp.int32)
    idx = _knn(yt8, byr, x8r, bxc, tlo, thi)
    idx_flat = idx.T.reshape(_NK)

    table = jnp.concatenate(
        [kv, xyz_i, jnp.full((_N, 1), t_i, f32), jnp.zeros((_N, 124), f32)], axis=1)
    g = _gather(table, idx_flat)

    xl8 = jnp.concatenate(
        [xyz_last, jnp.full((_N, 1), _T_LAST, f32), jnp.zeros((_N, 4), f32)], axis=1)
    pw1t8 = jnp.concatenate([p_w1.T, jnp.zeros((4, _PH), f32)], axis=0)
    pb1 = p_b1.reshape(1, _PH)

    st1 = _pe1stat(g, xl8, pw1t8, pb1)
    a1, c1 = _bn_ab(st1, _PH, p_g1, p_be1)

    wpre, vout, st2 = _passB(g, ql, xl8, pw1t8, pb1, a1, c1,
                             p_w2.T, p_b2.reshape(1, _C))
    ag1, cg1 = _bn_ab(st2, _C, w_g1, w_be1)

    h, st3 = _passC(wpre, ag1, cg1, w_w.T, w_b.reshape(1, _C))
    ag2, cg2 = _bn_ab(st3, _C, w_g2, w_be2)

    out3 = _passD(h.reshape(_N, _K, _C), vout.reshape(_N, _K, _C),
                  ag2.reshape(1, 1, _C), cg2.reshape(1, 1, _C))
    return out3.reshape(_N, _C)


# speedup vs baseline: 7.6533x; 1.0483x over previous
import functools

import jax
import jax.numpy as jnp
from jax import lax
from jax.experimental import pallas as pl
from jax.experimental.pallas import tpu as pltpu
from jax.experimental.pallas import tpu_sc as plsc

_N = 8192
_K = 16
_C = 256
_PH = 64
_T_LAST = 1.0
_EPS = 1e-5
_NK = _N * _K

_MASKVAL = 1e38

_RBP = 256


def _proj_body(fl_ref, fi_ref, qwt_ref, qb_ref, kwt_ref, kb_ref, vwt_ref, vb_ref,
               ql_ref, kv_ref):
    fl = fl_ref[...].astype(jnp.bfloat16)
    fi = fi_ref[...].astype(jnp.bfloat16)
    ql_ref[...] = jnp.dot(fl, qwt_ref[...].astype(jnp.bfloat16),
                          preferred_element_type=jnp.float32) + qb_ref[...]
    kv_ref[:, 0:_C] = jnp.dot(fi, kwt_ref[...].astype(jnp.bfloat16),
                              preferred_element_type=jnp.float32) + kb_ref[...]
    kv_ref[:, _C:2 * _C] = jnp.dot(fi, vwt_ref[...].astype(jnp.bfloat16),
                                   preferred_element_type=jnp.float32) + vb_ref[...]


def _proj(fea_last, fea_i, q_wt, q_b, k_wt, k_b, v_wt, v_b):
    grid = (_N // _RBP,)
    return pl.pallas_call(
        _proj_body,
        grid=grid,
        in_specs=[
            pl.BlockSpec((_RBP, _C), lambda b: (b, 0)),
            pl.BlockSpec((_RBP, _C), lambda b: (b, 0)),
            pl.BlockSpec((_C, _C), lambda b: (0, 0)),
            pl.BlockSpec((1, _C), lambda b: (0, 0)),
            pl.BlockSpec((_C, _C), lambda b: (0, 0)),
            pl.BlockSpec((1, _C), lambda b: (0, 0)),
            pl.BlockSpec((_C, _C), lambda b: (0, 0)),
            pl.BlockSpec((1, _C), lambda b: (0, 0)),
        ],
        out_specs=[
            pl.BlockSpec((_RBP, _C), lambda b: (b, 0)),
            pl.BlockSpec((_RBP, 2 * _C), lambda b: (b, 0)),
        ],
        out_shape=[
            jax.ShapeDtypeStruct((_N, _C), jnp.float32),
            jax.ShapeDtypeStruct((_N, 2 * _C), jnp.float32),
        ],
    )(fea_last, fea_i, q_wt, q_b, k_wt, k_b, v_wt, v_b)


_RB = 256
_CT = 512
_NT = _N // _CT


def _knn_body(tlo_ref, thi_ref, yt_ref, by_ref, xr_ref, bx_ref, idx_ref, d_ref):
    pid = pl.program_id(0)
    tlo = tlo_ref[pid]
    thi = thi_ref[pid]
    yt = yt_ref[...]
    ybt = yt.astype(jnp.bfloat16)
    by = by_ref[...]
    big = jnp.int32(2 ** 30)
    inf = jnp.float32(jnp.inf)
    yy = yt[0:1, :] * yt[0:1, :] + yt[1:2, :] * yt[1:2, :] + yt[2:3, :] * yt[2:3, :]

    am = jnp.full((1, _RB), -1, jnp.int32)
    for k in range(_K):
        am_prev = am

        def step(t, carry, am_prev=am_prev, first=(k == 0), last=(k == _K - 1)):
            m, am = carry
            sl = pl.ds(t * _CT, _CT)
            rowi = lax.broadcasted_iota(jnp.int32, (_CT, _RB), 0) + t * _CT
            if first:
                xs = xr_ref[sl, :]
                xx = (xs[:, 0:1] * xs[:, 0:1] + xs[:, 1:2] * xs[:, 1:2]
                      + xs[:, 2:3] * xs[:, 2:3])
                dot = jnp.dot(xs.astype(jnp.bfloat16), ybt,
                              preferred_element_type=jnp.float32)
                tile = yy + xx - 2.0 * dot
                bx = bx_ref[sl, :]
                tile = jnp.where(bx != by, jnp.float32(_MASKVAL), tile)
            else:
                tile = jnp.where(rowi == am_prev, inf, d_ref[sl, :])
            if not last:
                d_ref[sl, :] = tile
            tmin = jnp.min(tile, axis=0, keepdims=True)
            tam = jnp.min(jnp.where(tile <= tmin, rowi, big), axis=0, keepdims=True)
            upd = tmin < m
            return (jnp.where(upd, tmin, m), jnp.where(upd, tam, am))

        m, am = lax.fori_loop(
            tlo, thi, step,
            (jnp.full((1, _RB), inf, jnp.float32), jnp.full((1, _RB), big, jnp.int32)))
        idx_ref[k:k + 1, :] = am


def _knn(yt8, byr, x8r, bxc, tlo, thi):
    grid_spec = pltpu.PrefetchScalarGridSpec(
        num_scalar_prefetch=2,
        grid=(_N // _RB,),
        in_specs=[
            pl.BlockSpec((8, _RB), lambda b, *_: (0, b)),
            pl.BlockSpec((1, _RB), lambda b, *_: (0, b)),
            pl.BlockSpec((_N, 8), lambda b, *_: (0, 0)),
            pl.BlockSpec((_N, 1), lambda b, *_: (0, 0)),
        ],
        out_specs=pl.BlockSpec((_K, _RB), lambda b, *_: (0, b)),
        scratch_shapes=[pltpu.VMEM((_N, _RB), jnp.float32)],
    )
    return pl.pallas_call(
        _knn_body,
        grid_spec=grid_spec,
        out_shape=jax.ShapeDtypeStruct((_K, _N), jnp.int32),
    )(tlo, thi, yt8, byr, x8r, bxc)


_NW = 32
_BPW = _NK // _NW
_G = 128


_TW = 2 * _C + 128


def _gather(table, idx_flat):
    mesh = plsc.VectorSubcoreMesh(core_axis_name="c", subcore_axis_name="s")

    @functools.partial(
        pl.kernel,
        mesh=mesh,
        out_type=jax.ShapeDtypeStruct((_NK, _TW), jnp.float32),
        scratch_types=[
            pltpu.VMEM((_G,), jnp.int32),
            pltpu.VMEM((_G, _TW), jnp.float32),
            pltpu.SemaphoreType.DMA,
        ],
    )
    def gk(t_hbm, idx_hbm, o_hbm, idx_v, r1, sem):
        wid = lax.axis_index("s") * 2 + lax.axis_index("c")
        base = wid * _BPW

        def body(i, c):
            off = base + i * _G
            pltpu.sync_copy(idx_hbm.at[pl.ds(off, _G)], idx_v)
            pltpu.async_copy(t_hbm.at[idx_v], r1, sem).wait()
            pltpu.sync_copy(r1, o_hbm.at[pl.ds(off, _G)])
            return c

        lax.fori_loop(0, _BPW // _G, body, 0)

    return gk(table, idx_flat)


_FB = 512
_NP = _FB // _K


def _expand_mat():
    r = lax.broadcasted_iota(jnp.int32, (_FB, _NP), 0) // _K
    c = lax.broadcasted_iota(jnp.int32, (_FB, _NP), 1)
    return (r == c).astype(jnp.float32)


def _leaky(x):
    return jnp.where(x >= 0, x, 0.01 * x)


_FBS = 2048
_NPS = _FBS // _K


def _expand_mat_s():
    r = lax.broadcasted_iota(jnp.int32, (_FBS, _NPS), 0) // _K
    c = lax.broadcasted_iota(jnp.int32, (_FBS, _NPS), 1)
    return (r == c).astype(jnp.float32)


def _pe1stat_body(xg_ref, xl8_ref, pw1t_ref, pb1_ref, st_ref):
    e = _expand_mat_s()
    delta = xg_ref[:, 0:8] - jnp.dot(e, xl8_ref[...], preferred_element_type=jnp.float32)
    pe1 = jnp.dot(delta, pw1t_ref[...], preferred_element_type=jnp.float32) + pb1_ref[...]
    st_ref[0:1, 0:1, 0:_PH] = jnp.sum(pe1, axis=0, keepdims=True).reshape(1, 1, _PH)
    st_ref[0:1, 0:1, _PH:2 * _PH] = jnp.sum(pe1 * pe1, axis=0, keepdims=True).reshape(1, 1, _PH)


def _pe1stat(g, xl8, pw1t8, pb1):
    grid = (_NK // _FBS,)
    return pl.pallas_call(
        _pe1stat_body,
        grid=grid,
        in_specs=[
            pl.BlockSpec((_FBS, 128), lambda b: (b, 4)),
            pl.BlockSpec((_NPS, 8), lambda b: (b, 0)),
            pl.BlockSpec((8, _PH), lambda b: (0, 0)),
            pl.BlockSpec((1, _PH), lambda b: (0, 0)),
        ],
        out_specs=pl.BlockSpec((1, 1, 2 * _PH), lambda b: (b, 0, 0)),
        out_shape=jax.ShapeDtypeStruct((_NK // _FBS, 1, 2 * _PH), jnp.float32),
    )(g, xl8, pw1t8, pb1)


def _passB_body(xg_ref, kg_ref, vg_ref, ql_ref, xl8_ref, pw1t_ref, pb1_ref,
                a1_ref, c1_ref, pw2t_ref, pb2_ref,
                wpre_ref, vout_ref, st_ref):
    e = _expand_mat()
    delta = xg_ref[:, 0:8] - jnp.dot(e, xl8_ref[...], preferred_element_type=jnp.float32)
    pe1 = jnp.dot(delta, pw1t_ref[...], preferred_element_type=jnp.float32) + pb1_ref[...]
    pe1 = _leaky(pe1 * a1_ref[...] + c1_ref[...])
    pe2 = jnp.dot(pe1.astype(jnp.bfloat16), pw2t_ref[...].astype(jnp.bfloat16),
                  preferred_element_type=jnp.float32) + pb2_ref[...]
    qlr = jnp.dot(e, ql_ref[...], preferred_element_type=jnp.float32)
    wpre = qlr - kg_ref[...] + pe2
    wpre_ref[...] = wpre.astype(jnp.bfloat16)
    vout_ref[...] = (vg_ref[...] + pe2).astype(jnp.bfloat16)
    st_ref[0:1, 0:1, 0:_C] = jnp.sum(wpre, axis=0, keepdims=True).reshape(1, 1, _C)
    st_ref[0:1, 0:1, _C:2 * _C] = jnp.sum(wpre * wpre, axis=0, keepdims=True).reshape(1, 1, _C)


def _passB(g, ql, xl8, pw1t8, pb1, a1, c1, pw2t, pb2):
    grid = (_NK // _FB,)
    return pl.pallas_call(
        _passB_body,
        grid=grid,
        in_specs=[
            pl.BlockSpec((_FB, 128), lambda b: (b, 4)),
            pl.BlockSpec((_FB, _C), lambda b: (b, 0)),
            pl.BlockSpec((_FB, _C), lambda b: (b, 1)),
            pl.BlockSpec((_NP, _C), lambda b: (b, 0)),
            pl.BlockSpec((_NP, 8), lambda b: (b, 0)),
            pl.BlockSpec((8, _PH), lambda b: (0, 0)),
            pl.BlockSpec((1, _PH), lambda b: (0, 0)),
            pl.BlockSpec((1, _PH), lambda b: (0, 0)),
            pl.BlockSpec((1, _PH), lambda b: (0, 0)),
            pl.BlockSpec((_PH, _C), lambda b: (0, 0)),
            pl.BlockSpec((1, _C), lambda b: (0, 0)),
        ],
        out_specs=[
            pl.BlockSpec((_FB, _C), lambda b: (b, 0)),
            pl.BlockSpec((_FB, _C), lambda b: (b, 0)),
            pl.BlockSpec((1, 1, 2 * _C), lambda b: (b, 0, 0)),
        ],
        out_shape=[
            jax.ShapeDtypeStruct((_NK, _C), jnp.bfloat16),
            jax.ShapeDtypeStruct((_NK, _C), jnp.bfloat16),
            jax.ShapeDtypeStruct((_NK // _FB, 1, 2 * _C), jnp.float32),
        ],
    )(g, g, g, ql, xl8, pw1t8, pb1, a1, c1, pw2t, pb2)


def _passC_body(wpre_ref, ag_ref, cg_ref, wwt_ref, wb_ref, h_ref, st_ref):
    s = _leaky(wpre_ref[...].astype(jnp.float32) * ag_ref[...] + cg_ref[...])
    h = jnp.dot(s.astype(jnp.bfloat16), wwt_ref[...].astype(jnp.bfloat16),
                preferred_element_type=jnp.float32) + wb_ref[...]
    h_ref[...] = h
    st_ref[0:1, 0:1, 0:_C] = jnp.sum(h, axis=0, keepdims=True).reshape(1, 1, _C)
    st_ref[0:1, 0:1, _C:2 * _C] = jnp.sum(h * h, axis=0, keepdims=True).reshape(1, 1, _C)


def _passC(wpre, ag1, cg1, wwt, wb):
    grid = (_NK // _FB,)
    return pl.pallas_call(
        _passC_body,
        grid=grid,
        in_specs=[
            pl.BlockSpec((_FB, _C), lambda b: (b, 0)),
            pl.BlockSpec((1, _C), lambda b: (0, 0)),
            pl.BlockSpec((1, _C), lambda b: (0, 0)),
            pl.BlockSpec((_C, _C), lambda b: (0, 0)),
            pl.BlockSpec((1, _C), lambda b: (0, 0)),
        ],
        out_specs=[
            pl.BlockSpec((_FB, _C), lambda b: (b, 0)),
            pl.BlockSpec((1, 1, 2 * _C), lambda b: (b, 0, 0)),
        ],
        out_shape=[
            jax.ShapeDtypeStruct((_NK, _C), jnp.float32),
            jax.ShapeDtypeStruct((_NK // _FB, 1, 2 * _C), jnp.float32),
        ],
    )(wpre, ag1, cg1, wwt, wb)


def _tree_red(x, op):
    w = _K
    while w > 1:
        h = w // 2
        x = op(x[:, 0:h], x[:, h:w])
        w = h
    return x


def _passD_body(h_ref, v_ref, ag_ref, cg_ref, out_ref):
    s = _leaky(h_ref[...] * ag_ref[...] + cg_ref[...])
    m = _tree_red(s, jnp.maximum)
    ex = jnp.exp(s - m)
    den = _tree_red(ex, jnp.add)
    w = ex / den
    out_ref[...] = _tree_red(w * v_ref[...].astype(jnp.float32), jnp.add)


_NPD = 64


def _passD(h3, v3, ag2, cg2):
    grid = (_N // _NPD,)
    return pl.pallas_call(
        _passD_body,
        grid=grid,
        in_specs=[
            pl.BlockSpec((_NPD, _K, _C), lambda b: (b, 0, 0)),
            pl.BlockSpec((_NPD, _K, _C), lambda b: (b, 0, 0)),
            pl.BlockSpec((1, 1, _C), lambda b: (0, 0, 0)),
            pl.BlockSpec((1, 1, _C), lambda b: (0, 0, 0)),
        ],
        out_specs=pl.BlockSpec((_NPD, 1, _C), lambda b: (b, 0, 0)),
        out_shape=jax.ShapeDtypeStruct((_N, 1, _C), jnp.float32),
    )(h3, v3, ag2, cg2)


def _bn_ab(stats3, width, gamma, beta):
    stats = stats3.reshape(-1, 2 * width)
    s1 = jnp.sum(stats[:, 0:width], axis=0)
    s2 = jnp.sum(stats[:, width:2 * width], axis=0)
    mean = s1 / _NK
    var = s2 / _NK - mean * mean
    a = gamma / jnp.sqrt(var + _EPS)
    b = beta - mean * a
    return a.reshape(1, width), b.reshape(1, width)


def kernel(fea_i, fea_last, xyz_i, xyz_last, batch, t_i,
           p_w1, p_b1, p_g1, p_be1, p_w2, p_b2,
           q_w, q_b, k_w, k_b, v_w, v_b,
           w_g1, w_be1, w_w, w_b, w_g2, w_be2):
    f32 = jnp.float32
    t_i = jnp.asarray(t_i, f32)

    ql, kv = _proj(fea_last, fea_i,
                   q_w.T, q_b.reshape(1, _C), k_w.T, k_b.reshape(1, _C),
                   v_w.T, v_b.reshape(1, _C))

    byr = batch.reshape(1, _N)
    bxc = batch.reshape(_N, 1)
    yt8 = jnp.concatenate([xyz_last.T, jnp.zeros((5, _N), f32)], axis=0)
    x8r = jnp.concatenate([xyz_i, jnp.zeros((_N, 5), f32)], axis=1)
    bounds = jnp.searchsorted(batch, jnp.arange(9, dtype=jnp.int32),
                              side="left").astype(jnp.int32)
    bf = batch[0::_RB]
    bl = batch[_RB - 1::_RB]
    tlo = (bounds[bf] // _CT).astype(jnp.int32)
    thi = ((bounds[bl + 1] + _CT - 1) // _CT).astype(jnp.int32)
    idx = _knn(yt8, byr, x8r, bxc, tlo, thi)
    idx_flat = idx.T.reshape(_NK)

    table = jnp.concatenate(
        [kv, xyz_i, jnp.full((_N, 1), t_i, f32), jnp.zeros((_N, 124), f32)], axis=1)
    g = _gather(table, idx_flat)

    xl8 = jnp.concatenate(
        [xyz_last, jnp.full((_N, 1), _T_LAST, f32), jnp.zeros((_N, 4), f32)], axis=1)
    pw1t8 = jnp.concatenate([p_w1.T, jnp.zeros((4, _PH), f32)], axis=0)
    pb1 = p_b1.reshape(1, _PH)

    st1 = _pe1stat(g, xl8, pw1t8, pb1)
    a1, c1 = _bn_ab(st1, _PH, p_g1, p_be1)

    wpre, vout, st2 = _passB(g, ql, xl8, pw1t8, pb1, a1, c1,
                             p_w2.T, p_b2.reshape(1, _C))
    ag1, cg1 = _bn_ab(st2, _C, w_g1, w_be1)

    h, st3 = _passC(wpre, ag1, cg1, w_w.T, w_b.reshape(1, _C))
    ag2, cg2 = _bn_ab(st3, _C, w_g2, w_be2)

    out3 = _passD(h.reshape(_N, _K, _C), vout.reshape(_N, _K, _C),
                  ag2.reshape(1, 1, _C), cg2.reshape(1, 1, _C))
    return out3.reshape(_N, _C)


# speedup vs baseline: 7.7753x; 1.0159x over previous
import functools

import jax
import jax.numpy as jnp
from jax import lax
from jax.experimental import pallas as pl
from jax.experimental.pallas import tpu as pltpu
from jax.experimental.pallas import tpu_sc as plsc

_N = 8192
_K = 16
_C = 256
_PH = 64
_T_LAST = 1.0
_EPS = 1e-5
_NK = _N * _K

_MASKVAL = 1e38

_RBP = 256


def _proj_body(fl_ref, fi_ref, qwt_ref, qb_ref, kwt_ref, kb_ref, vwt_ref, vb_ref,
               ql_ref, kv_ref):
    fl = fl_ref[...].astype(jnp.bfloat16)
    fi = fi_ref[...].astype(jnp.bfloat16)
    ql_ref[...] = jnp.dot(fl, qwt_ref[...].astype(jnp.bfloat16),
                          preferred_element_type=jnp.float32) + qb_ref[...]
    kv_ref[:, 0:_C] = jnp.dot(fi, kwt_ref[...].astype(jnp.bfloat16),
                              preferred_element_type=jnp.float32) + kb_ref[...]
    kv_ref[:, _C:2 * _C] = jnp.dot(fi, vwt_ref[...].astype(jnp.bfloat16),
                                   preferred_element_type=jnp.float32) + vb_ref[...]


def _proj(fea_last, fea_i, q_wt, q_b, k_wt, k_b, v_wt, v_b):
    grid = (_N // _RBP,)
    return pl.pallas_call(
        _proj_body,
        grid=grid,
        in_specs=[
            pl.BlockSpec((_RBP, _C), lambda b: (b, 0)),
            pl.BlockSpec((_RBP, _C), lambda b: (b, 0)),
            pl.BlockSpec((_C, _C), lambda b: (0, 0)),
            pl.BlockSpec((1, _C), lambda b: (0, 0)),
            pl.BlockSpec((_C, _C), lambda b: (0, 0)),
            pl.BlockSpec((1, _C), lambda b: (0, 0)),
            pl.BlockSpec((_C, _C), lambda b: (0, 0)),
            pl.BlockSpec((1, _C), lambda b: (0, 0)),
        ],
        out_specs=[
            pl.BlockSpec((_RBP, _C), lambda b: (b, 0)),
            pl.BlockSpec((_RBP, 2 * _C), lambda b: (b, 0)),
        ],
        out_shape=[
            jax.ShapeDtypeStruct((_N, _C), jnp.float32),
            jax.ShapeDtypeStruct((_N, 2 * _C), jnp.float32),
        ],
    )(fea_last, fea_i, q_wt, q_b, k_wt, k_b, v_wt, v_b)


_RB = 256
_CT = 512
_NT = _N // _CT


def _knn_body(tlo_ref, thi_ref, yt_ref, by_ref, xr_ref, bx_ref, idx_ref, d_ref):
    pid = pl.program_id(0)
    tlo = tlo_ref[pid]
    thi = thi_ref[pid]
    yt = yt_ref[...]
    ybt = yt.astype(jnp.bfloat16)
    by = by_ref[...]
    big = jnp.int32(2 ** 30)
    inf = jnp.float32(jnp.inf)
    yy = yt[0:1, :] * yt[0:1, :] + yt[1:2, :] * yt[1:2, :] + yt[2:3, :] * yt[2:3, :]

    am = jnp.full((1, _RB), -1, jnp.int32)
    for k in range(_K):
        am_prev = am

        def step(t, carry, am_prev=am_prev, first=(k == 0), last=(k == _K - 1)):
            m, am = carry
            sl = pl.ds(t * _CT, _CT)
            rowi = lax.broadcasted_iota(jnp.int32, (_CT, _RB), 0) + t * _CT
            if first:
                xs = xr_ref[sl, :]
                xx = (xs[:, 0:1] * xs[:, 0:1] + xs[:, 1:2] * xs[:, 1:2]
                      + xs[:, 2:3] * xs[:, 2:3])
                dot = jnp.dot(xs.astype(jnp.bfloat16), ybt,
                              preferred_element_type=jnp.float32)
                tile = yy + xx - 2.0 * dot
                bx = bx_ref[sl, :]
                tile = jnp.where(bx != by, jnp.float32(_MASKVAL), tile)
            else:
                tile = jnp.where(rowi == am_prev, inf, d_ref[sl, :])
            if not last:
                d_ref[sl, :] = tile
            tmin = jnp.min(tile, axis=0, keepdims=True)
            tam = jnp.min(jnp.where(tile <= tmin, rowi, big), axis=0, keepdims=True)
            upd = tmin < m
            return (jnp.where(upd, tmin, m), jnp.where(upd, tam, am))

        m, am = lax.fori_loop(
            tlo, thi, step,
            (jnp.full((1, _RB), inf, jnp.float32), jnp.full((1, _RB), big, jnp.int32)))
        idx_ref[k:k + 1, :] = am


def _knn(yt8, byr, x8r, bxc, tlo, thi):
    grid_spec = pltpu.PrefetchScalarGridSpec(
        num_scalar_prefetch=2,
        grid=(_N // _RB,),
        in_specs=[
            pl.BlockSpec((8, _RB), lambda b, *_: (0, b)),
            pl.BlockSpec((1, _RB), lambda b, *_: (0, b)),
            pl.BlockSpec((_N, 8), lambda b, *_: (0, 0)),
            pl.BlockSpec((_N, 1), lambda b, *_: (0, 0)),
        ],
        out_specs=pl.BlockSpec((_K, _RB), lambda b, *_: (0, b)),
        scratch_shapes=[pltpu.VMEM((_N, _RB), jnp.float32)],
    )
    return pl.pallas_call(
        _knn_body,
        grid_spec=grid_spec,
        out_shape=jax.ShapeDtypeStruct((_K, _N), jnp.int32),
    )(tlo, thi, yt8, byr, x8r, bxc)


_NW = 32
_BPW = _NK // _NW
_G = 128


_TW = 2 * _C + 128


def _gather(table, idx_flat):
    mesh = plsc.VectorSubcoreMesh(core_axis_name="c", subcore_axis_name="s")

    @functools.partial(
        pl.kernel,
        mesh=mesh,
        out_type=jax.ShapeDtypeStruct((_NK, _TW), jnp.float32),
        scratch_types=[
            pltpu.VMEM((_G,), jnp.int32),
            pltpu.VMEM((_G, _TW), jnp.float32),
            pltpu.SemaphoreType.DMA,
        ],
    )
    def gk(t_hbm, idx_hbm, o_hbm, idx_v, r1, sem):
        wid = lax.axis_index("s") * 2 + lax.axis_index("c")
        base = wid * _BPW

        def body(i, c):
            off = base + i * _G
            pltpu.sync_copy(idx_hbm.at[pl.ds(off, _G)], idx_v)
            pltpu.async_copy(t_hbm.at[idx_v], r1, sem).wait()
            pltpu.sync_copy(r1, o_hbm.at[pl.ds(off, _G)])
            return c

        lax.fori_loop(0, _BPW // _G, body, 0)

    return gk(table, idx_flat)


_FB = 512
_NP = _FB // _K


def _expand_mat():
    r = lax.broadcasted_iota(jnp.int32, (_FB, _NP), 0) // _K
    c = lax.broadcasted_iota(jnp.int32, (_FB, _NP), 1)
    return (r == c).astype(jnp.float32)


def _leaky(x):
    return jnp.where(x >= 0, x, 0.01 * x)


_FBS = 2048
_NPS = _FBS // _K


def _expand_mat_s():
    r = lax.broadcasted_iota(jnp.int32, (_FBS, _NPS), 0) // _K
    c = lax.broadcasted_iota(jnp.int32, (_FBS, _NPS), 1)
    return (r == c).astype(jnp.float32)


def _pe1stat_body(xg_ref, xl8_ref, pw1t_ref, pb1_ref, st_ref):
    e = _expand_mat_s()
    delta = xg_ref[:, 0:8] - jnp.dot(e, xl8_ref[...], preferred_element_type=jnp.float32)
    pe1 = jnp.dot(delta, pw1t_ref[...], preferred_element_type=jnp.float32) + pb1_ref[...]
    st_ref[0:1, 0:1, 0:_PH] = jnp.sum(pe1, axis=0, keepdims=True).reshape(1, 1, _PH)
    st_ref[0:1, 0:1, _PH:2 * _PH] = jnp.sum(pe1 * pe1, axis=0, keepdims=True).reshape(1, 1, _PH)


def _pe1stat(g, xl8, pw1t8, pb1):
    grid = (_NK // _FBS,)
    return pl.pallas_call(
        _pe1stat_body,
        grid=grid,
        in_specs=[
            pl.BlockSpec((_FBS, 128), lambda b: (b, 4)),
            pl.BlockSpec((_NPS, 8), lambda b: (b, 0)),
            pl.BlockSpec((8, _PH), lambda b: (0, 0)),
            pl.BlockSpec((1, _PH), lambda b: (0, 0)),
        ],
        out_specs=pl.BlockSpec((1, 1, 2 * _PH), lambda b: (b, 0, 0)),
        out_shape=jax.ShapeDtypeStruct((_NK // _FBS, 1, 2 * _PH), jnp.float32),
    )(g, xl8, pw1t8, pb1)


def _passB_body(xg_ref, kg_ref, vg_ref, ql_ref, xl8_ref, pw1t_ref, pb1_ref,
                a1_ref, c1_ref, pw2t_ref, pb2_ref,
                wpre_ref, vout_ref, st_ref):
    e = _expand_mat()
    delta = xg_ref[:, 0:8] - jnp.dot(e, xl8_ref[...], preferred_element_type=jnp.float32)
    pe1 = jnp.dot(delta, pw1t_ref[...], preferred_element_type=jnp.float32) + pb1_ref[...]
    pe1 = _leaky(pe1 * a1_ref[...] + c1_ref[...])
    pe2 = jnp.dot(pe1.astype(jnp.bfloat16), pw2t_ref[...].astype(jnp.bfloat16),
                  preferred_element_type=jnp.float32) + pb2_ref[...]
    qlr = jnp.dot(e, ql_ref[...], preferred_element_type=jnp.float32)
    wpre = qlr - kg_ref[...] + pe2
    wpre_ref[...] = wpre.astype(jnp.bfloat16)
    vout_ref[...] = (vg_ref[...] + pe2).astype(jnp.bfloat16)
    st_ref[0:1, 0:1, 0:_C] = jnp.sum(wpre, axis=0, keepdims=True).reshape(1, 1, _C)
    st_ref[0:1, 0:1, _C:2 * _C] = jnp.sum(wpre * wpre, axis=0, keepdims=True).reshape(1, 1, _C)


def _passB(g, ql, xl8, pw1t8, pb1, a1, c1, pw2t, pb2):
    grid = (_NK // _FB,)
    return pl.pallas_call(
        _passB_body,
        grid=grid,
        in_specs=[
            pl.BlockSpec((_FB, 128), lambda b: (b, 4)),
            pl.BlockSpec((_FB, _C), lambda b: (b, 0)),
            pl.BlockSpec((_FB, _C), lambda b: (b, 1)),
            pl.BlockSpec((_NP, _C), lambda b: (b, 0)),
            pl.BlockSpec((_NP, 8), lambda b: (b, 0)),
            pl.BlockSpec((8, _PH), lambda b: (0, 0)),
            pl.BlockSpec((1, _PH), lambda b: (0, 0)),
            pl.BlockSpec((1, _PH), lambda b: (0, 0)),
            pl.BlockSpec((1, _PH), lambda b: (0, 0)),
            pl.BlockSpec((_PH, _C), lambda b: (0, 0)),
            pl.BlockSpec((1, _C), lambda b: (0, 0)),
        ],
        out_specs=[
            pl.BlockSpec((_FB, _C), lambda b: (b, 0)),
            pl.BlockSpec((_FB, _C), lambda b: (b, 0)),
            pl.BlockSpec((1, 1, 2 * _C), lambda b: (b, 0, 0)),
        ],
        out_shape=[
            jax.ShapeDtypeStruct((_NK, _C), jnp.bfloat16),
            jax.ShapeDtypeStruct((_NK, _C), jnp.bfloat16),
            jax.ShapeDtypeStruct((_NK // _FB, 1, 2 * _C), jnp.float32),
        ],
    )(g, g, g, ql, xl8, pw1t8, pb1, a1, c1, pw2t, pb2)


def _passC_body(wpre_ref, ag_ref, cg_ref, wwt_ref, wb_ref, h_ref, st_ref):
    s = _leaky(wpre_ref[...].astype(jnp.float32) * ag_ref[...] + cg_ref[...])
    h = jnp.dot(s.astype(jnp.bfloat16), wwt_ref[...].astype(jnp.bfloat16),
                preferred_element_type=jnp.float32) + wb_ref[...]
    h_ref[...] = h.astype(jnp.bfloat16)
    st_ref[0:1, 0:1, 0:_C] = jnp.sum(h, axis=0, keepdims=True).reshape(1, 1, _C)
    st_ref[0:1, 0:1, _C:2 * _C] = jnp.sum(h * h, axis=0, keepdims=True).reshape(1, 1, _C)


def _passC(wpre, ag1, cg1, wwt, wb):
    grid = (_NK // _FB,)
    return pl.pallas_call(
        _passC_body,
        grid=grid,
        in_specs=[
            pl.BlockSpec((_FB, _C), lambda b: (b, 0)),
            pl.BlockSpec((1, _C), lambda b: (0, 0)),
            pl.BlockSpec((1, _C), lambda b: (0, 0)),
            pl.BlockSpec((_C, _C), lambda b: (0, 0)),
            pl.BlockSpec((1, _C), lambda b: (0, 0)),
        ],
        out_specs=[
            pl.BlockSpec((_FB, _C), lambda b: (b, 0)),
            pl.BlockSpec((1, 1, 2 * _C), lambda b: (b, 0, 0)),
        ],
        out_shape=[
            jax.ShapeDtypeStruct((_NK, _C), jnp.bfloat16),
            jax.ShapeDtypeStruct((_NK // _FB, 1, 2 * _C), jnp.float32),
        ],
    )(wpre, ag1, cg1, wwt, wb)


def _tree_red(x, op):
    w = _K
    while w > 1:
        h = w // 2
        x = op(x[:, 0:h], x[:, h:w])
        w = h
    return x


def _passD_body(h_ref, v_ref, ag_ref, cg_ref, out_ref):
    s = _leaky(h_ref[...].astype(jnp.float32) * ag_ref[...] + cg_ref[...])
    m = _tree_red(s, jnp.maximum)
    ex = jnp.exp(s - m)
    den = _tree_red(ex, jnp.add)
    w = ex / den
    out_ref[...] = _tree_red(w * v_ref[...].astype(jnp.float32), jnp.add)


_NPD = 64


def _passD(h3, v3, ag2, cg2):
    grid = (_N // _NPD,)
    return pl.pallas_call(
        _passD_body,
        grid=grid,
        in_specs=[
            pl.BlockSpec((_NPD, _K, _C), lambda b: (b, 0, 0)),
            pl.BlockSpec((_NPD, _K, _C), lambda b: (b, 0, 0)),
            pl.BlockSpec((1, 1, _C), lambda b: (0, 0, 0)),
            pl.BlockSpec((1, 1, _C), lambda b: (0, 0, 0)),
        ],
        out_specs=pl.BlockSpec((_NPD, 1, _C), lambda b: (b, 0, 0)),
        out_shape=jax.ShapeDtypeStruct((_N, 1, _C), jnp.float32),
    )(h3, v3, ag2, cg2)


def _bn_ab(stats3, width, gamma, beta):
    stats = stats3.reshape(-1, 2 * width)
    s1 = jnp.sum(stats[:, 0:width], axis=0)
    s2 = jnp.sum(stats[:, width:2 * width], axis=0)
    mean = s1 / _NK
    var = s2 / _NK - mean * mean
    a = gamma / jnp.sqrt(var + _EPS)
    b = beta - mean * a
    return a.reshape(1, width), b.reshape(1, width)


def kernel(fea_i, fea_last, xyz_i, xyz_last, batch, t_i,
           p_w1, p_b1, p_g1, p_be1, p_w2, p_b2,
           q_w, q_b, k_w, k_b, v_w, v_b,
           w_g1, w_be1, w_w, w_b, w_g2, w_be2):
    f32 = jnp.float32
    t_i = jnp.asarray(t_i, f32)

    ql, kv = _proj(fea_last, fea_i,
                   q_w.T, q_b.reshape(1, _C), k_w.T, k_b.reshape(1, _C),
                   v_w.T, v_b.reshape(1, _C))

    byr = batch.reshape(1, _N)
    bxc = batch.reshape(_N, 1)
    yt8 = jnp.concatenate([xyz_last.T, jnp.zeros((5, _N), f32)], axis=0)
    x8r = jnp.concatenate([xyz_i, jnp.zeros((_N, 5), f32)], axis=1)
    bounds = jnp.searchsorted(batch, jnp.arange(9, dtype=jnp.int32),
                              side="left").astype(jnp.int32)
    bf = batch[0::_RB]
    bl = batch[_RB - 1::_RB]
    tlo = (bounds[bf] // _CT).astype(jnp.int32)
    thi = ((bounds[bl + 1] + _CT - 1) // _CT).astype(jnp.int32)
    idx = _knn(yt8, byr, x8r, bxc, tlo, thi)
    idx_flat = idx.T.reshape(_NK)

    table = jnp.concatenate(
        [kv, xyz_i, jnp.full((_N, 1), t_i, f32), jnp.zeros((_N, 124), f32)], axis=1)
    g = _gather(table, idx_flat)

    xl8 = jnp.concatenate(
        [xyz_last, jnp.full((_N, 1), _T_LAST, f32), jnp.zeros((_N, 4), f32)], axis=1)
    pw1t8 = jnp.concatenate([p_w1.T, jnp.zeros((4, _PH), f32)], axis=0)
    pb1 = p_b1.reshape(1, _PH)

    st1 = _pe1stat(g, xl8, pw1t8, pb1)
    a1, c1 = _bn_ab(st1, _PH, p_g1, p_be1)

    wpre, vout, st2 = _passB(g, ql, xl8, pw1t8, pb1, a1, c1,
                             p_w2.T, p_b2.reshape(1, _C))
    ag1, cg1 = _bn_ab(st2, _C, w_g1, w_be1)

    h, st3 = _passC(wpre, ag1, cg1, w_w.T, w_b.reshape(1, _C))
    ag2, cg2 = _bn_ab(st3, _C, w_g2, w_be2)

    out3 = _passD(h.reshape(_N, _K, _C), vout.reshape(_N, _K, _C),
                  ag2.reshape(1, 1, _C), cg2.reshape(1, 1, _C))
    return out3.reshape(_N, _C)


# speedup vs baseline: 7.8640x; 1.0114x over previous
import functools

import jax
import jax.numpy as jnp
from jax import lax
from jax.experimental import pallas as pl
from jax.experimental.pallas import tpu as pltpu
from jax.experimental.pallas import tpu_sc as plsc

_N = 8192
_K = 16
_C = 256
_PH = 64
_T_LAST = 1.0
_EPS = 1e-5
_NK = _N * _K

_MASKVAL = 1e38

_RBP = 256


def _proj_body(fl_ref, fi_ref, qwt_ref, qb_ref, kwt_ref, kb_ref, vwt_ref, vb_ref,
               ql_ref, kv_ref):
    fl = fl_ref[...].astype(jnp.bfloat16)
    fi = fi_ref[...].astype(jnp.bfloat16)
    ql_ref[...] = jnp.dot(fl, qwt_ref[...].astype(jnp.bfloat16),
                          preferred_element_type=jnp.float32) + qb_ref[...]
    kv_ref[:, 0:_C] = jnp.dot(fi, kwt_ref[...].astype(jnp.bfloat16),
                              preferred_element_type=jnp.float32) + kb_ref[...]
    kv_ref[:, _C:2 * _C] = jnp.dot(fi, vwt_ref[...].astype(jnp.bfloat16),
                                   preferred_element_type=jnp.float32) + vb_ref[...]


def _proj(fea_last, fea_i, q_wt, q_b, k_wt, k_b, v_wt, v_b):
    grid = (_N // _RBP,)
    return pl.pallas_call(
        _proj_body,
        grid=grid,
        in_specs=[
            pl.BlockSpec((_RBP, _C), lambda b: (b, 0)),
            pl.BlockSpec((_RBP, _C), lambda b: (b, 0)),
            pl.BlockSpec((_C, _C), lambda b: (0, 0)),
            pl.BlockSpec((1, _C), lambda b: (0, 0)),
            pl.BlockSpec((_C, _C), lambda b: (0, 0)),
            pl.BlockSpec((1, _C), lambda b: (0, 0)),
            pl.BlockSpec((_C, _C), lambda b: (0, 0)),
            pl.BlockSpec((1, _C), lambda b: (0, 0)),
        ],
        out_specs=[
            pl.BlockSpec((_RBP, _C), lambda b: (b, 0)),
            pl.BlockSpec((_RBP, 2 * _C), lambda b: (b, 0)),
        ],
        out_shape=[
            jax.ShapeDtypeStruct((_N, _C), jnp.float32),
            jax.ShapeDtypeStruct((_N, 2 * _C), jnp.float32),
        ],
    )(fea_last, fea_i, q_wt, q_b, k_wt, k_b, v_wt, v_b)


_RB = 256
_CT = 512
_NT = _N // _CT


def _knn_body(tlo_ref, thi_ref, yt_ref, by_ref, xr_ref, bx_ref, idx_ref, d_ref):
    pid = pl.program_id(0)
    tlo = tlo_ref[pid]
    thi = thi_ref[pid]
    yt = yt_ref[...]
    ybt = yt.astype(jnp.bfloat16)
    by = by_ref[...]
    big = jnp.int32(2 ** 30)
    inf = jnp.float32(jnp.inf)
    yy = yt[0:1, :] * yt[0:1, :] + yt[1:2, :] * yt[1:2, :] + yt[2:3, :] * yt[2:3, :]

    am = jnp.full((1, _RB), -1, jnp.int32)
    for k in range(_K):
        am_prev = am

        def step(t, carry, am_prev=am_prev, first=(k == 0), last=(k == _K - 1)):
            m, am = carry
            sl = pl.ds(t * _CT, _CT)
            rowi = lax.broadcasted_iota(jnp.int32, (_CT, _RB), 0) + t * _CT
            if first:
                xs = xr_ref[sl, :]
                xx = (xs[:, 0:1] * xs[:, 0:1] + xs[:, 1:2] * xs[:, 1:2]
                      + xs[:, 2:3] * xs[:, 2:3])
                dot = jnp.dot(xs.astype(jnp.bfloat16), ybt,
                              preferred_element_type=jnp.float32)
                tile = yy + xx - 2.0 * dot
                bx = bx_ref[sl, :]
                tile = jnp.where(bx != by, jnp.float32(_MASKVAL), tile)
            else:
                tile = jnp.where(rowi == am_prev, inf, d_ref[sl, :])
            if not last:
                d_ref[sl, :] = tile
            tmin = jnp.min(tile, axis=0, keepdims=True)
            tam = jnp.min(jnp.where(tile <= tmin, rowi, big), axis=0, keepdims=True)
            upd = tmin < m
            return (jnp.where(upd, tmin, m), jnp.where(upd, tam, am))

        m, am = lax.fori_loop(
            tlo, thi, step,
            (jnp.full((1, _RB), inf, jnp.float32), jnp.full((1, _RB), big, jnp.int32)))
        idx_ref[k:k + 1, :] = am


def _knn(yt8, byr, x8r, bxc, tlo, thi):
    grid_spec = pltpu.PrefetchScalarGridSpec(
        num_scalar_prefetch=2,
        grid=(_N // _RB,),
        in_specs=[
            pl.BlockSpec((8, _RB), lambda b, *_: (0, b)),
            pl.BlockSpec((1, _RB), lambda b, *_: (0, b)),
            pl.BlockSpec((_N, 8), lambda b, *_: (0, 0)),
            pl.BlockSpec((_N, 1), lambda b, *_: (0, 0)),
        ],
        out_specs=pl.BlockSpec((_K, _RB), lambda b, *_: (0, b)),
        scratch_shapes=[pltpu.VMEM((_N, _RB), jnp.float32)],
    )
    return pl.pallas_call(
        _knn_body,
        grid_spec=grid_spec,
        out_shape=jax.ShapeDtypeStruct((_K, _N), jnp.int32),
    )(tlo, thi, yt8, byr, x8r, bxc)


_NW = 32
_BPW = _NK // _NW
_G = 64


_TW = 2 * _C + 128


def _gather(table, idx_flat):
    mesh = plsc.VectorSubcoreMesh(core_axis_name="c", subcore_axis_name="s")

    @functools.partial(
        pl.kernel,
        mesh=mesh,
        out_type=jax.ShapeDtypeStruct((_NK, _TW), jnp.float32),
        scratch_types=[
            pltpu.VMEM((_G,), jnp.int32),
            pltpu.VMEM((_G,), jnp.int32),
            pltpu.VMEM((_G, _TW), jnp.float32),
            pltpu.VMEM((_G, _TW), jnp.float32),
            pltpu.SemaphoreType.DMA,
            pltpu.SemaphoreType.DMA,
        ],
    )
    def gk(t_hbm, idx_hbm, o_hbm, i0, i1, r0, r1, sem0, sem1):
        wid = lax.axis_index("s") * 2 + lax.axis_index("c")
        base = wid * _BPW
        n2 = _BPW // _G // 2

        def start(g, iv, rv, sem):
            pltpu.sync_copy(idx_hbm.at[pl.ds(base + g * _G, _G)], iv)
            pltpu.async_copy(t_hbm.at[iv], rv, sem)

        def drain(rv, sem):
            pltpu.make_async_copy(o_hbm.at[pl.ds(base, _G)], rv, sem).wait()

        start(0, i0, r0, sem0)
        start(1, i1, r1, sem1)

        def body(jj, c):
            g0 = jj * 2
            drain(r0, sem0)
            pltpu.sync_copy(r0, o_hbm.at[pl.ds(base + g0 * _G, _G)])

            @pl.when(jj < n2 - 1)
            def _():
                start(g0 + 2, i0, r0, sem0)

            drain(r1, sem1)
            pltpu.sync_copy(r1, o_hbm.at[pl.ds(base + (g0 + 1) * _G, _G)])

            @pl.when(jj < n2 - 1)
            def _():
                start(g0 + 3, i1, r1, sem1)

            return c

        lax.fori_loop(0, n2, body, 0)

    return gk(table, idx_flat)


_FB = 512
_NP = _FB // _K


def _expand_mat():
    r = lax.broadcasted_iota(jnp.int32, (_FB, _NP), 0) // _K
    c = lax.broadcasted_iota(jnp.int32, (_FB, _NP), 1)
    return (r == c).astype(jnp.float32)


def _leaky(x):
    return jnp.where(x >= 0, x, 0.01 * x)


_FBS = 2048
_NPS = _FBS // _K


def _expand_mat_s():
    r = lax.broadcasted_iota(jnp.int32, (_FBS, _NPS), 0) // _K
    c = lax.broadcasted_iota(jnp.int32, (_FBS, _NPS), 1)
    return (r == c).astype(jnp.float32)


def _pe1stat_body(xg_ref, xl8_ref, pw1t_ref, pb1_ref, st_ref):
    e = _expand_mat_s()
    delta = xg_ref[:, 0:8] - jnp.dot(e, xl8_ref[...], preferred_element_type=jnp.float32)
    pe1 = jnp.dot(delta, pw1t_ref[...], preferred_element_type=jnp.float32) + pb1_ref[...]
    st_ref[0:1, 0:1, 0:_PH] = jnp.sum(pe1, axis=0, keepdims=True).reshape(1, 1, _PH)
    st_ref[0:1, 0:1, _PH:2 * _PH] = jnp.sum(pe1 * pe1, axis=0, keepdims=True).reshape(1, 1, _PH)


def _pe1stat(g, xl8, pw1t8, pb1):
    grid = (_NK // _FBS,)
    return pl.pallas_call(
        _pe1stat_body,
        grid=grid,
        in_specs=[
            pl.BlockSpec((_FBS, 128), lambda b: (b, 4)),
            pl.BlockSpec((_NPS, 8), lambda b: (b, 0)),
            pl.BlockSpec((8, _PH), lambda b: (0, 0)),
            pl.BlockSpec((1, _PH), lambda b: (0, 0)),
        ],
        out_specs=pl.BlockSpec((1, 1, 2 * _PH), lambda b: (b, 0, 0)),
        out_shape=jax.ShapeDtypeStruct((_NK // _FBS, 1, 2 * _PH), jnp.float32),
    )(g, xl8, pw1t8, pb1)


def _passB_body(xg_ref, kg_ref, vg_ref, ql_ref, xl8_ref, pw1t_ref, pb1_ref,
                a1_ref, c1_ref, pw2t_ref, pb2_ref,
                wpre_ref, vout_ref, st_ref):
    e = _expand_mat()
    delta = xg_ref[:, 0:8] - jnp.dot(e, xl8_ref[...], preferred_element_type=jnp.float32)
    pe1 = jnp.dot(delta, pw1t_ref[...], preferred_element_type=jnp.float32) + pb1_ref[...]
    pe1 = _leaky(pe1 * a1_ref[...] + c1_ref[...])
    pe2 = jnp.dot(pe1.astype(jnp.bfloat16), pw2t_ref[...].astype(jnp.bfloat16),
                  preferred_element_type=jnp.float32) + pb2_ref[...]
    qlr = jnp.dot(e, ql_ref[...], preferred_element_type=jnp.float32)
    wpre = qlr - kg_ref[...] + pe2
    wpre_ref[...] = wpre.astype(jnp.bfloat16)
    vout_ref[...] = (vg_ref[...] + pe2).astype(jnp.bfloat16)
    st_ref[0:1, 0:1, 0:_C] = jnp.sum(wpre, axis=0, keepdims=True).reshape(1, 1, _C)
    st_ref[0:1, 0:1, _C:2 * _C] = jnp.sum(wpre * wpre, axis=0, keepdims=True).reshape(1, 1, _C)


def _passB(g, ql, xl8, pw1t8, pb1, a1, c1, pw2t, pb2):
    grid = (_NK // _FB,)
    return pl.pallas_call(
        _passB_body,
        grid=grid,
        in_specs=[
            pl.BlockSpec((_FB, 128), lambda b: (b, 4)),
            pl.BlockSpec((_FB, _C), lambda b: (b, 0)),
            pl.BlockSpec((_FB, _C), lambda b: (b, 1)),
            pl.BlockSpec((_NP, _C), lambda b: (b, 0)),
            pl.BlockSpec((_NP, 8), lambda b: (b, 0)),
            pl.BlockSpec((8, _PH), lambda b: (0, 0)),
            pl.BlockSpec((1, _PH), lambda b: (0, 0)),
            pl.BlockSpec((1, _PH), lambda b: (0, 0)),
            pl.BlockSpec((1, _PH), lambda b: (0, 0)),
            pl.BlockSpec((_PH, _C), lambda b: (0, 0)),
            pl.BlockSpec((1, _C), lambda b: (0, 0)),
        ],
        out_specs=[
            pl.BlockSpec((_FB, _C), lambda b: (b, 0)),
            pl.BlockSpec((_FB, _C), lambda b: (b, 0)),
            pl.BlockSpec((1, 1, 2 * _C), lambda b: (b, 0, 0)),
        ],
        out_shape=[
            jax.ShapeDtypeStruct((_NK, _C), jnp.bfloat16),
            jax.ShapeDtypeStruct((_NK, _C), jnp.bfloat16),
            jax.ShapeDtypeStruct((_NK // _FB, 1, 2 * _C), jnp.float32),
        ],
    )(g, g, g, ql, xl8, pw1t8, pb1, a1, c1, pw2t, pb2)


def _passC_body(wpre_ref, ag_ref, cg_ref, wwt_ref, wb_ref, h_ref, st_ref):
    s = _leaky(wpre_ref[...].astype(jnp.float32) * ag_ref[...] + cg_ref[...])
    h = jnp.dot(s.astype(jnp.bfloat16), wwt_ref[...].astype(jnp.bfloat16),
                preferred_element_type=jnp.float32) + wb_ref[...]
    h_ref[...] = h.astype(jnp.bfloat16)
    st_ref[0:1, 0:1, 0:_C] = jnp.sum(h, axis=0, keepdims=True).reshape(1, 1, _C)
    st_ref[0:1, 0:1, _C:2 * _C] = jnp.sum(h * h, axis=0, keepdims=True).reshape(1, 1, _C)


def _passC(wpre, ag1, cg1, wwt, wb):
    grid = (_NK // _FB,)
    return pl.pallas_call(
        _passC_body,
        grid=grid,
        in_specs=[
            pl.BlockSpec((_FB, _C), lambda b: (b, 0)),
            pl.BlockSpec((1, _C), lambda b: (0, 0)),
            pl.BlockSpec((1, _C), lambda b: (0, 0)),
            pl.BlockSpec((_C, _C), lambda b: (0, 0)),
            pl.BlockSpec((1, _C), lambda b: (0, 0)),
        ],
        out_specs=[
            pl.BlockSpec((_FB, _C), lambda b: (b, 0)),
            pl.BlockSpec((1, 1, 2 * _C), lambda b: (b, 0, 0)),
        ],
        out_shape=[
            jax.ShapeDtypeStruct((_NK, _C), jnp.bfloat16),
            jax.ShapeDtypeStruct((_NK // _FB, 1, 2 * _C), jnp.float32),
        ],
    )(wpre, ag1, cg1, wwt, wb)


def _tree_red(x, op):
    w = _K
    while w > 1:
        h = w // 2
        x = op(x[:, 0:h], x[:, h:w])
        w = h
    return x


def _passD_body(h_ref, v_ref, ag_ref, cg_ref, out_ref):
    s = _leaky(h_ref[...].astype(jnp.float32) * ag_ref[...] + cg_ref[...])
    m = _tree_red(s, jnp.maximum)
    ex = jnp.exp(s - m)
    den = _tree_red(ex, jnp.add)
    w = ex / den
    out_ref[...] = _tree_red(w * v_ref[...].astype(jnp.float32), jnp.add)


_NPD = 64


def _passD(h3, v3, ag2, cg2):
    grid = (_N // _NPD,)
    return pl.pallas_call(
        _passD_body,
        grid=grid,
        in_specs=[
            pl.BlockSpec((_NPD, _K, _C), lambda b: (b, 0, 0)),
            pl.BlockSpec((_NPD, _K, _C), lambda b: (b, 0, 0)),
            pl.BlockSpec((1, 1, _C), lambda b: (0, 0, 0)),
            pl.BlockSpec((1, 1, _C), lambda b: (0, 0, 0)),
        ],
        out_specs=pl.BlockSpec((_NPD, 1, _C), lambda b: (b, 0, 0)),
        out_shape=jax.ShapeDtypeStruct((_N, 1, _C), jnp.float32),
    )(h3, v3, ag2, cg2)


def _bn_ab(stats3, width, gamma, beta):
    stats = stats3.reshape(-1, 2 * width)
    s1 = jnp.sum(stats[:, 0:width], axis=0)
    s2 = jnp.sum(stats[:, width:2 * width], axis=0)
    mean = s1 / _NK
    var = s2 / _NK - mean * mean
    a = gamma / jnp.sqrt(var + _EPS)
    b = beta - mean * a
    return a.reshape(1, width), b.reshape(1, width)


def kernel(fea_i, fea_last, xyz_i, xyz_last, batch, t_i,
           p_w1, p_b1, p_g1, p_be1, p_w2, p_b2,
           q_w, q_b, k_w, k_b, v_w, v_b,
           w_g1, w_be1, w_w, w_b, w_g2, w_be2):
    f32 = jnp.float32
    t_i = jnp.asarray(t_i, f32)

    ql, kv = _proj(fea_last, fea_i,
                   q_w.T, q_b.reshape(1, _C), k_w.T, k_b.reshape(1, _C),
                   v_w.T, v_b.reshape(1, _C))

    byr = batch.reshape(1, _N)
    bxc = batch.reshape(_N, 1)
    yt8 = jnp.concatenate([xyz_last.T, jnp.zeros((5, _N), f32)], axis=0)
    x8r = jnp.concatenate([xyz_i, jnp.zeros((_N, 5), f32)], axis=1)
    bounds = jnp.searchsorted(batch, jnp.arange(9, dtype=jnp.int32),
                              side="left").astype(jnp.int32)
    bf = batch[0::_RB]
    bl = batch[_RB - 1::_RB]
    tlo = (bounds[bf] // _CT).astype(jnp.int32)
    thi = ((bounds[bl + 1] + _CT - 1) // _CT).astype(jnp.int32)
    idx = _knn(yt8, byr, x8r, bxc, tlo, thi)
    idx_flat = idx.T.reshape(_NK)

    table = jnp.concatenate(
        [kv, xyz_i, jnp.full((_N, 1), t_i, f32), jnp.zeros((_N, 124), f32)], axis=1)
    g = _gather(table, idx_flat)

    xl8 = jnp.concatenate(
        [xyz_last, jnp.full((_N, 1), _T_LAST, f32), jnp.zeros((_N, 4), f32)], axis=1)
    pw1t8 = jnp.concatenate([p_w1.T, jnp.zeros((4, _PH), f32)], axis=0)
    pb1 = p_b1.reshape(1, _PH)

    st1 = _pe1stat(g, xl8, pw1t8, pb1)
    a1, c1 = _bn_ab(st1, _PH, p_g1, p_be1)

    wpre, vout, st2 = _passB(g, ql, xl8, pw1t8, pb1, a1, c1,
                             p_w2.T, p_b2.reshape(1, _C))
    ag1, cg1 = _bn_ab(st2, _C, w_g1, w_be1)

    h, st3 = _passC(wpre, ag1, cg1, w_w.T, w_b.reshape(1, _C))
    ag2, cg2 = _bn_ab(st3, _C, w_g2, w_be2)

    out3 = _passD(h.reshape(_N, _K, _C), vout.reshape(_N, _K, _C),
                  ag2.reshape(1, 1, _C), cg2.reshape(1, 1, _C))
    return out3.reshape(_N, _C)


# speedup vs baseline: 8.7128x; 1.1079x over previous
import functools

import jax
import jax.numpy as jnp
from jax import lax
from jax.experimental import pallas as pl
from jax.experimental.pallas import tpu as pltpu
from jax.experimental.pallas import tpu_sc as plsc

_N = 8192
_K = 16
_C = 256
_PH = 64
_T_LAST = 1.0
_EPS = 1e-5
_NK = _N * _K

_MASKVAL = 1e38

_RBP = 256


def _rn_bf16_bits(u):
    half = jnp.int32(0x7FFF) + jnp.bitwise_and(lax.shift_right_logical(u, 16), 1)
    return jnp.bitwise_and(u + half, jnp.int32(-65536))


def _proj_body(fl_ref, fi_ref, qwt_ref, qb_ref, kwt_ref, kb_ref, vwt_ref, vb_ref,
               ql_ref, kv_ref):
    fl = fl_ref[...].astype(jnp.bfloat16)
    fi = fi_ref[...].astype(jnp.bfloat16)
    ql_ref[...] = jnp.dot(fl, qwt_ref[...].astype(jnp.bfloat16),
                          preferred_element_type=jnp.float32) + qb_ref[...]
    k = jnp.dot(fi, kwt_ref[...].astype(jnp.bfloat16),
                preferred_element_type=jnp.float32) + kb_ref[...]
    v = jnp.dot(fi, vwt_ref[...].astype(jnp.bfloat16),
                preferred_element_type=jnp.float32) + vb_ref[...]
    ku = _rn_bf16_bits(lax.bitcast_convert_type(k, jnp.int32))
    vu = _rn_bf16_bits(lax.bitcast_convert_type(v, jnp.int32))
    packed = jnp.bitwise_or(ku, lax.shift_right_logical(vu, 16))
    kv_ref[...] = lax.bitcast_convert_type(packed, jnp.float32)


def _proj(fea_last, fea_i, q_wt, q_b, k_wt, k_b, v_wt, v_b):
    grid = (_N // _RBP,)
    return pl.pallas_call(
        _proj_body,
        grid=grid,
        in_specs=[
            pl.BlockSpec((_RBP, _C), lambda b: (b, 0)),
            pl.BlockSpec((_RBP, _C), lambda b: (b, 0)),
            pl.BlockSpec((_C, _C), lambda b: (0, 0)),
            pl.BlockSpec((1, _C), lambda b: (0, 0)),
            pl.BlockSpec((_C, _C), lambda b: (0, 0)),
            pl.BlockSpec((1, _C), lambda b: (0, 0)),
            pl.BlockSpec((_C, _C), lambda b: (0, 0)),
            pl.BlockSpec((1, _C), lambda b: (0, 0)),
        ],
        out_specs=[
            pl.BlockSpec((_RBP, _C), lambda b: (b, 0)),
            pl.BlockSpec((_RBP, _C), lambda b: (b, 0)),
        ],
        out_shape=[
            jax.ShapeDtypeStruct((_N, _C), jnp.float32),
            jax.ShapeDtypeStruct((_N, _C), jnp.float32),
        ],
    )(fea_last, fea_i, q_wt, q_b, k_wt, k_b, v_wt, v_b)


_RB = 256
_CT = 512
_NT = _N // _CT


def _knn_body(tlo_ref, thi_ref, yt_ref, by_ref, xr_ref, bx_ref, idx_ref, d_ref):
    pid = pl.program_id(0)
    tlo = tlo_ref[pid]
    thi = thi_ref[pid]
    yt = yt_ref[...]
    ybt = yt.astype(jnp.bfloat16)
    by = by_ref[...]
    big = jnp.int32(2 ** 30)
    inf = jnp.float32(jnp.inf)
    yy = yt[0:1, :] * yt[0:1, :] + yt[1:2, :] * yt[1:2, :] + yt[2:3, :] * yt[2:3, :]

    am = jnp.full((1, _RB), -1, jnp.int32)
    for k in range(_K):
        am_prev = am

        def step(t, carry, am_prev=am_prev, first=(k == 0), last=(k == _K - 1)):
            m, am = carry
            sl = pl.ds(t * _CT, _CT)
            rowi = lax.broadcasted_iota(jnp.int32, (_CT, _RB), 0) + t * _CT
            if first:
                xs = xr_ref[sl, :]
                xx = (xs[:, 0:1] * xs[:, 0:1] + xs[:, 1:2] * xs[:, 1:2]
                      + xs[:, 2:3] * xs[:, 2:3])
                dot = jnp.dot(xs.astype(jnp.bfloat16), ybt,
                              preferred_element_type=jnp.float32)
                tile = yy + xx - 2.0 * dot
                bx = bx_ref[sl, :]
                tile = jnp.where(bx != by, jnp.float32(_MASKVAL), tile)
            else:
                tile = jnp.where(rowi == am_prev, inf, d_ref[sl, :])
            if not last:
                d_ref[sl, :] = tile
            tmin = jnp.min(tile, axis=0, keepdims=True)
            tam = jnp.min(jnp.where(tile <= tmin, rowi, big), axis=0, keepdims=True)
            upd = tmin < m
            return (jnp.where(upd, tmin, m), jnp.where(upd, tam, am))

        m, am = lax.fori_loop(
            tlo, thi, step,
            (jnp.full((1, _RB), inf, jnp.float32), jnp.full((1, _RB), big, jnp.int32)))
        idx_ref[k:k + 1, :] = am


def _knn(yt8, byr, x8r, bxc, tlo, thi):
    grid_spec = pltpu.PrefetchScalarGridSpec(
        num_scalar_prefetch=2,
        grid=(_N // _RB,),
        in_specs=[
            pl.BlockSpec((8, _RB), lambda b, *_: (0, b)),
            pl.BlockSpec((1, _RB), lambda b, *_: (0, b)),
            pl.BlockSpec((_N, 8), lambda b, *_: (0, 0)),
            pl.BlockSpec((_N, 1), lambda b, *_: (0, 0)),
        ],
        out_specs=pl.BlockSpec((_K, _RB), lambda b, *_: (0, b)),
        scratch_shapes=[pltpu.VMEM((_N, _RB), jnp.float32)],
    )
    return pl.pallas_call(
        _knn_body,
        grid_spec=grid_spec,
        out_shape=jax.ShapeDtypeStruct((_K, _N), jnp.int32),
    )(tlo, thi, yt8, byr, x8r, bxc)


_NW = 32
_BPW = _NK // _NW
_G = 64


_TW = _C + 128


def _gather(table, idx_flat):
    mesh = plsc.VectorSubcoreMesh(core_axis_name="c", subcore_axis_name="s")

    @functools.partial(
        pl.kernel,
        mesh=mesh,
        out_type=jax.ShapeDtypeStruct((_NK, _TW), jnp.float32),
        scratch_types=[
            pltpu.VMEM((_G,), jnp.int32),
            pltpu.VMEM((_G,), jnp.int32),
            pltpu.VMEM((_G, _TW), jnp.float32),
            pltpu.VMEM((_G, _TW), jnp.float32),
            pltpu.SemaphoreType.DMA,
            pltpu.SemaphoreType.DMA,
        ],
    )
    def gk(t_hbm, idx_hbm, o_hbm, i0, i1, r0, r1, sem0, sem1):
        wid = lax.axis_index("s") * 2 + lax.axis_index("c")
        base = wid * _BPW
        n2 = _BPW // _G // 2

        def start(g, iv, rv, sem):
            pltpu.sync_copy(idx_hbm.at[pl.ds(base + g * _G, _G)], iv)
            pltpu.async_copy(t_hbm.at[iv], rv, sem)

        def drain(rv, sem):
            pltpu.make_async_copy(o_hbm.at[pl.ds(base, _G)], rv, sem).wait()

        start(0, i0, r0, sem0)
        start(1, i1, r1, sem1)

        def body(jj, c):
            g0 = jj * 2
            drain(r0, sem0)
            pltpu.sync_copy(r0, o_hbm.at[pl.ds(base + g0 * _G, _G)])

            @pl.when(jj < n2 - 1)
            def _():
                start(g0 + 2, i0, r0, sem0)

            drain(r1, sem1)
            pltpu.sync_copy(r1, o_hbm.at[pl.ds(base + (g0 + 1) * _G, _G)])

            @pl.when(jj < n2 - 1)
            def _():
                start(g0 + 3, i1, r1, sem1)

            return c

        lax.fori_loop(0, n2, body, 0)

    return gk(table, idx_flat)


_FB = 512
_NP = _FB // _K


def _expand_mat():
    r = lax.broadcasted_iota(jnp.int32, (_FB, _NP), 0) // _K
    c = lax.broadcasted_iota(jnp.int32, (_FB, _NP), 1)
    return (r == c).astype(jnp.float32)


def _leaky(x):
    return jnp.where(x >= 0, x, 0.01 * x)


_FBS = 2048
_NPS = _FBS // _K


def _expand_mat_s():
    r = lax.broadcasted_iota(jnp.int32, (_FBS, _NPS), 0) // _K
    c = lax.broadcasted_iota(jnp.int32, (_FBS, _NPS), 1)
    return (r == c).astype(jnp.float32)


def _pe1stat_body(xg_ref, xl8_ref, pw1t_ref, pb1_ref, st_ref):
    e = _expand_mat_s()
    delta = xg_ref[:, 0:8] - jnp.dot(e, xl8_ref[...], preferred_element_type=jnp.float32)
    pe1 = jnp.dot(delta, pw1t_ref[...], preferred_element_type=jnp.float32) + pb1_ref[...]
    st_ref[0:1, 0:1, 0:_PH] = jnp.sum(pe1, axis=0, keepdims=True).reshape(1, 1, _PH)
    st_ref[0:1, 0:1, _PH:2 * _PH] = jnp.sum(pe1 * pe1, axis=0, keepdims=True).reshape(1, 1, _PH)


def _pe1stat(g, xl8, pw1t8, pb1):
    grid = (_NK // _FBS,)
    return pl.pallas_call(
        _pe1stat_body,
        grid=grid,
        in_specs=[
            pl.BlockSpec((_FBS, 128), lambda b: (b, 2)),
            pl.BlockSpec((_NPS, 8), lambda b: (b, 0)),
            pl.BlockSpec((8, _PH), lambda b: (0, 0)),
            pl.BlockSpec((1, _PH), lambda b: (0, 0)),
        ],
        out_specs=pl.BlockSpec((1, 1, 2 * _PH), lambda b: (b, 0, 0)),
        out_shape=jax.ShapeDtypeStruct((_NK // _FBS, 1, 2 * _PH), jnp.float32),
    )(g, xl8, pw1t8, pb1)


def _passB_body(xg_ref, kvg_ref, ql_ref, xl8_ref, pw1t_ref, pb1_ref,
                a1_ref, c1_ref, pw2t_ref, pb2_ref,
                wpre_ref, vout_ref, st_ref):
    u = lax.bitcast_convert_type(kvg_ref[...], jnp.int32)
    kg = lax.bitcast_convert_type(jnp.bitwise_and(u, jnp.int32(-65536)), jnp.float32)
    vg = lax.bitcast_convert_type(lax.shift_left(u, 16), jnp.float32)
    e = _expand_mat()
    delta = xg_ref[:, 0:8] - jnp.dot(e, xl8_ref[...], preferred_element_type=jnp.float32)
    pe1 = jnp.dot(delta, pw1t_ref[...], preferred_element_type=jnp.float32) + pb1_ref[...]
    pe1 = _leaky(pe1 * a1_ref[...] + c1_ref[...])
    pe2 = jnp.dot(pe1.astype(jnp.bfloat16), pw2t_ref[...].astype(jnp.bfloat16),
                  preferred_element_type=jnp.float32) + pb2_ref[...]
    qlr = jnp.dot(e, ql_ref[...], preferred_element_type=jnp.float32)
    wpre = qlr - kg + pe2
    wpre_ref[...] = wpre.astype(jnp.bfloat16)
    vout_ref[...] = (vg + pe2).astype(jnp.bfloat16)
    st_ref[0:1, 0:1, 0:_C] = jnp.sum(wpre, axis=0, keepdims=True).reshape(1, 1, _C)
    st_ref[0:1, 0:1, _C:2 * _C] = jnp.sum(wpre * wpre, axis=0, keepdims=True).reshape(1, 1, _C)


def _passB(g, ql, xl8, pw1t8, pb1, a1, c1, pw2t, pb2):
    grid = (_NK // _FB,)
    return pl.pallas_call(
        _passB_body,
        grid=grid,
        in_specs=[
            pl.BlockSpec((_FB, 128), lambda b: (b, 2)),
            pl.BlockSpec((_FB, _C), lambda b: (b, 0)),
            pl.BlockSpec((_NP, _C), lambda b: (b, 0)),
            pl.BlockSpec((_NP, 8), lambda b: (b, 0)),
            pl.BlockSpec((8, _PH), lambda b: (0, 0)),
            pl.BlockSpec((1, _PH), lambda b: (0, 0)),
            pl.BlockSpec((1, _PH), lambda b: (0, 0)),
            pl.BlockSpec((1, _PH), lambda b: (0, 0)),
            pl.BlockSpec((_PH, _C), lambda b: (0, 0)),
            pl.BlockSpec((1, _C), lambda b: (0, 0)),
        ],
        out_specs=[
            pl.BlockSpec((_FB, _C), lambda b: (b, 0)),
            pl.BlockSpec((_FB, _C), lambda b: (b, 0)),
            pl.BlockSpec((1, 1, 2 * _C), lambda b: (b, 0, 0)),
        ],
        out_shape=[
            jax.ShapeDtypeStruct((_NK, _C), jnp.bfloat16),
            jax.ShapeDtypeStruct((_NK, _C), jnp.bfloat16),
            jax.ShapeDtypeStruct((_NK // _FB, 1, 2 * _C), jnp.float32),
        ],
    )(g, g, ql, xl8, pw1t8, pb1, a1, c1, pw2t, pb2)


def _passC_body(wpre_ref, ag_ref, cg_ref, wwt_ref, wb_ref, h_ref, st_ref):
    s = _leaky(wpre_ref[...].astype(jnp.float32) * ag_ref[...] + cg_ref[...])
    h = jnp.dot(s.astype(jnp.bfloat16), wwt_ref[...].astype(jnp.bfloat16),
                preferred_element_type=jnp.float32) + wb_ref[...]
    h_ref[...] = h.astype(jnp.bfloat16)
    st_ref[0:1, 0:1, 0:_C] = jnp.sum(h, axis=0, keepdims=True).reshape(1, 1, _C)
    st_ref[0:1, 0:1, _C:2 * _C] = jnp.sum(h * h, axis=0, keepdims=True).reshape(1, 1, _C)


def _passC(wpre, ag1, cg1, wwt, wb):
    grid = (_NK // _FB,)
    return pl.pallas_call(
        _passC_body,
        grid=grid,
        in_specs=[
            pl.BlockSpec((_FB, _C), lambda b: (b, 0)),
            pl.BlockSpec((1, _C), lambda b: (0, 0)),
            pl.BlockSpec((1, _C), lambda b: (0, 0)),
            pl.BlockSpec((_C, _C), lambda b: (0, 0)),
            pl.BlockSpec((1, _C), lambda b: (0, 0)),
        ],
        out_specs=[
            pl.BlockSpec((_FB, _C), lambda b: (b, 0)),
            pl.BlockSpec((1, 1, 2 * _C), lambda b: (b, 0, 0)),
        ],
        out_shape=[
            jax.ShapeDtypeStruct((_NK, _C), jnp.bfloat16),
            jax.ShapeDtypeStruct((_NK // _FB, 1, 2 * _C), jnp.float32),
        ],
    )(wpre, ag1, cg1, wwt, wb)


def _tree_red(x, op):
    w = _K
    while w > 1:
        h = w // 2
        x = op(x[:, 0:h], x[:, h:w])
        w = h
    return x


def _passD_body(h_ref, v_ref, ag_ref, cg_ref, out_ref):
    s = _leaky(h_ref[...].astype(jnp.float32) * ag_ref[...] + cg_ref[...])
    m = _tree_red(s, jnp.maximum)
    ex = jnp.exp(s - m)
    den = _tree_red(ex, jnp.add)
    w = ex / den
    out_ref[...] = _tree_red(w * v_ref[...].astype(jnp.float32), jnp.add)


_NPD = 64


def _passD(h3, v3, ag2, cg2):
    grid = (_N // _NPD,)
    return pl.pallas_call(
        _passD_body,
        grid=grid,
        in_specs=[
            pl.BlockSpec((_NPD, _K, _C), lambda b: (b, 0, 0)),
            pl.BlockSpec((_NPD, _K, _C), lambda b: (b, 0, 0)),
            pl.BlockSpec((1, 1, _C), lambda b: (0, 0, 0)),
            pl.BlockSpec((1, 1, _C), lambda b: (0, 0, 0)),
        ],
        out_specs=pl.BlockSpec((_NPD, 1, _C), lambda b: (b, 0, 0)),
        out_shape=jax.ShapeDtypeStruct((_N, 1, _C), jnp.float32),
    )(h3, v3, ag2, cg2)


def _bn_ab(stats3, width, gamma, beta):
    stats = stats3.reshape(-1, 2 * width)
    s1 = jnp.sum(stats[:, 0:width], axis=0)
    s2 = jnp.sum(stats[:, width:2 * width], axis=0)
    mean = s1 / _NK
    var = s2 / _NK - mean * mean
    a = gamma / jnp.sqrt(var + _EPS)
    b = beta - mean * a
    return a.reshape(1, width), b.reshape(1, width)


def kernel(fea_i, fea_last, xyz_i, xyz_last, batch, t_i,
           p_w1, p_b1, p_g1, p_be1, p_w2, p_b2,
           q_w, q_b, k_w, k_b, v_w, v_b,
           w_g1, w_be1, w_w, w_b, w_g2, w_be2):
    f32 = jnp.float32
    t_i = jnp.asarray(t_i, f32)

    ql, kv = _proj(fea_last, fea_i,
                   q_w.T, q_b.reshape(1, _C), k_w.T, k_b.reshape(1, _C),
                   v_w.T, v_b.reshape(1, _C))

    byr = batch.reshape(1, _N)
    bxc = batch.reshape(_N, 1)
    yt8 = jnp.concatenate([xyz_last.T, jnp.zeros((5, _N), f32)], axis=0)
    x8r = jnp.concatenate([xyz_i, jnp.zeros((_N, 5), f32)], axis=1)
    bounds = jnp.searchsorted(batch, jnp.arange(9, dtype=jnp.int32),
                              side="left").astype(jnp.int32)
    bf = batch[0::_RB]
    bl = batch[_RB - 1::_RB]
    tlo = (bounds[bf] // _CT).astype(jnp.int32)
    thi = ((bounds[bl + 1] + _CT - 1) // _CT).astype(jnp.int32)
    idx = _knn(yt8, byr, x8r, bxc, tlo, thi)
    idx_flat = idx.T.reshape(_NK)

    table = jnp.concatenate(
        [kv, xyz_i, jnp.full((_N, 1), t_i, f32), jnp.zeros((_N, 124), f32)], axis=1)
    g = _gather(table, idx_flat)

    xl8 = jnp.concatenate(
        [xyz_last, jnp.full((_N, 1), _T_LAST, f32), jnp.zeros((_N, 4), f32)], axis=1)
    pw1t8 = jnp.concatenate([p_w1.T, jnp.zeros((4, _PH), f32)], axis=0)
    pb1 = p_b1.reshape(1, _PH)

    st1 = _pe1stat(g, xl8, pw1t8, pb1)
    a1, c1 = _bn_ab(st1, _PH, p_g1, p_be1)

    wpre, vout, st2 = _passB(g, ql, xl8, pw1t8, pb1, a1, c1,
                             p_w2.T, p_b2.reshape(1, _C))
    ag1, cg1 = _bn_ab(st2, _C, w_g1, w_be1)

    h, st3 = _passC(wpre, ag1, cg1, w_w.T, w_b.reshape(1, _C))
    ag2, cg2 = _bn_ab(st3, _C, w_g2, w_be2)

    out3 = _passD(h.reshape(_N, _K, _C), vout.reshape(_N, _K, _C),
                  ag2.reshape(1, 1, _C), cg2.reshape(1, 1, _C))
    return out3.reshape(_N, _C)


# speedup vs baseline: 8.7198x; 1.0008x over previous
import functools

import jax
import jax.numpy as jnp
from jax import lax
from jax.experimental import pallas as pl
from jax.experimental.pallas import tpu as pltpu
from jax.experimental.pallas import tpu_sc as plsc

_N = 8192
_K = 16
_C = 256
_PH = 64
_T_LAST = 1.0
_EPS = 1e-5
_NK = _N * _K

_MASKVAL = 1e38

_RBP = 256


def _rn_bf16_bits(u):
    half = jnp.int32(0x7FFF) + jnp.bitwise_and(lax.shift_right_logical(u, 16), 1)
    return jnp.bitwise_and(u + half, jnp.int32(-65536))


def _proj_body(fl_ref, fi_ref, qwt_ref, qb_ref, kwt_ref, kb_ref, vwt_ref, vb_ref,
               ql_ref, kv_ref):
    fl = fl_ref[...].astype(jnp.bfloat16)
    fi = fi_ref[...].astype(jnp.bfloat16)
    ql_ref[...] = jnp.dot(fl, qwt_ref[...].astype(jnp.bfloat16),
                          preferred_element_type=jnp.float32) + qb_ref[...]
    k = jnp.dot(fi, kwt_ref[...].astype(jnp.bfloat16),
                preferred_element_type=jnp.float32) + kb_ref[...]
    v = jnp.dot(fi, vwt_ref[...].astype(jnp.bfloat16),
                preferred_element_type=jnp.float32) + vb_ref[...]
    ku = _rn_bf16_bits(lax.bitcast_convert_type(k, jnp.int32))
    vu = _rn_bf16_bits(lax.bitcast_convert_type(v, jnp.int32))
    packed = jnp.bitwise_or(ku, lax.shift_right_logical(vu, 16))
    kv_ref[...] = lax.bitcast_convert_type(packed, jnp.float32)


def _proj(fea_last, fea_i, q_wt, q_b, k_wt, k_b, v_wt, v_b):
    grid = (_N // _RBP,)
    return pl.pallas_call(
        _proj_body,
        grid=grid,
        in_specs=[
            pl.BlockSpec((_RBP, _C), lambda b: (b, 0)),
            pl.BlockSpec((_RBP, _C), lambda b: (b, 0)),
            pl.BlockSpec((_C, _C), lambda b: (0, 0)),
            pl.BlockSpec((1, _C), lambda b: (0, 0)),
            pl.BlockSpec((_C, _C), lambda b: (0, 0)),
            pl.BlockSpec((1, _C), lambda b: (0, 0)),
            pl.BlockSpec((_C, _C), lambda b: (0, 0)),
            pl.BlockSpec((1, _C), lambda b: (0, 0)),
        ],
        out_specs=[
            pl.BlockSpec((_RBP, _C), lambda b: (b, 0)),
            pl.BlockSpec((_RBP, _C), lambda b: (b, 0)),
        ],
        out_shape=[
            jax.ShapeDtypeStruct((_N, _C), jnp.float32),
            jax.ShapeDtypeStruct((_N, _C), jnp.float32),
        ],
    )(fea_last, fea_i, q_wt, q_b, k_wt, k_b, v_wt, v_b)


_RB = 256
_CT = 256
_NT = _N // _CT


def _knn_body(tlo_ref, thi_ref, yt_ref, by_ref, xr_ref, bx_ref, idx_ref, d_ref):
    pid = pl.program_id(0)
    tlo = tlo_ref[pid]
    thi = thi_ref[pid]
    yt = yt_ref[...]
    ybt = yt.astype(jnp.bfloat16)
    by = by_ref[...]
    big = jnp.int32(2 ** 30)
    inf = jnp.float32(jnp.inf)
    yy = yt[0:1, :] * yt[0:1, :] + yt[1:2, :] * yt[1:2, :] + yt[2:3, :] * yt[2:3, :]

    am = jnp.full((1, _RB), -1, jnp.int32)
    for k in range(_K):
        am_prev = am

        def step(t, carry, am_prev=am_prev, first=(k == 0), last=(k == _K - 1)):
            m, am = carry
            sl = pl.ds(t * _CT, _CT)
            rowi = lax.broadcasted_iota(jnp.int32, (_CT, _RB), 0) + t * _CT
            if first:
                xs = xr_ref[sl, :]
                xx = (xs[:, 0:1] * xs[:, 0:1] + xs[:, 1:2] * xs[:, 1:2]
                      + xs[:, 2:3] * xs[:, 2:3])
                dot = jnp.dot(xs.astype(jnp.bfloat16), ybt,
                              preferred_element_type=jnp.float32)
                tile = yy + xx - 2.0 * dot
                bx = bx_ref[sl, :]
                tile = jnp.where(bx != by, jnp.float32(_MASKVAL), tile)
            else:
                tile = jnp.where(rowi == am_prev, inf, d_ref[sl, :])
            if not last:
                d_ref[sl, :] = tile
            tmin = jnp.min(tile, axis=0, keepdims=True)
            tam = jnp.min(jnp.where(tile <= tmin, rowi, big), axis=0, keepdims=True)
            upd = tmin < m
            return (jnp.where(upd, tmin, m), jnp.where(upd, tam, am))

        m, am = lax.fori_loop(
            tlo, thi, step,
            (jnp.full((1, _RB), inf, jnp.float32), jnp.full((1, _RB), big, jnp.int32)))
        idx_ref[k:k + 1, :] = am


def _knn(yt8, byr, x8r, bxc, tlo, thi):
    grid_spec = pltpu.PrefetchScalarGridSpec(
        num_scalar_prefetch=2,
        grid=(_N // _RB,),
        in_specs=[
            pl.BlockSpec((8, _RB), lambda b, *_: (0, b)),
            pl.BlockSpec((1, _RB), lambda b, *_: (0, b)),
            pl.BlockSpec((_N, 8), lambda b, *_: (0, 0)),
            pl.BlockSpec((_N, 1), lambda b, *_: (0, 0)),
        ],
        out_specs=pl.BlockSpec((_K, _RB), lambda b, *_: (0, b)),
        scratch_shapes=[pltpu.VMEM((_N, _RB), jnp.float32)],
    )
    return pl.pallas_call(
        _knn_body,
        grid_spec=grid_spec,
        out_shape=jax.ShapeDtypeStruct((_K, _N), jnp.int32),
    )(tlo, thi, yt8, byr, x8r, bxc)


_NW = 32
_BPW = _NK // _NW
_G = 64


_TW = _C + 128


def _gather(table, idx_flat):
    mesh = plsc.VectorSubcoreMesh(core_axis_name="c", subcore_axis_name="s")

    @functools.partial(
        pl.kernel,
        mesh=mesh,
        out_type=jax.ShapeDtypeStruct((_NK, _TW), jnp.float32),
        scratch_types=[
            pltpu.VMEM((_G,), jnp.int32),
            pltpu.VMEM((_G,), jnp.int32),
            pltpu.VMEM((_G, _TW), jnp.float32),
            pltpu.VMEM((_G, _TW), jnp.float32),
            pltpu.SemaphoreType.DMA,
            pltpu.SemaphoreType.DMA,
        ],
    )
    def gk(t_hbm, idx_hbm, o_hbm, i0, i1, r0, r1, sem0, sem1):
        wid = lax.axis_index("s") * 2 + lax.axis_index("c")
        base = wid * _BPW
        n2 = _BPW // _G // 2

        def start(g, iv, rv, sem):
            pltpu.sync_copy(idx_hbm.at[pl.ds(base + g * _G, _G)], iv)
            pltpu.async_copy(t_hbm.at[iv], rv, sem)

        def drain(rv, sem):
            pltpu.make_async_copy(o_hbm.at[pl.ds(base, _G)], rv, sem).wait()

        start(0, i0, r0, sem0)
        start(1, i1, r1, sem1)

        def body(jj, c):
            g0 = jj * 2
            drain(r0, sem0)
            pltpu.sync_copy(r0, o_hbm.at[pl.ds(base + g0 * _G, _G)])

            @pl.when(jj < n2 - 1)
            def _():
                start(g0 + 2, i0, r0, sem0)

            drain(r1, sem1)
            pltpu.sync_copy(r1, o_hbm.at[pl.ds(base + (g0 + 1) * _G, _G)])

            @pl.when(jj < n2 - 1)
            def _():
                start(g0 + 3, i1, r1, sem1)

            return c

        lax.fori_loop(0, n2, body, 0)

    return gk(table, idx_flat)


_FB = 512
_NP = _FB // _K


def _expand_mat():
    r = lax.broadcasted_iota(jnp.int32, (_FB, _NP), 0) // _K
    c = lax.broadcasted_iota(jnp.int32, (_FB, _NP), 1)
    return (r == c).astype(jnp.float32)


def _leaky(x):
    return jnp.where(x >= 0, x, 0.01 * x)


_FBS = 2048
_NPS = _FBS // _K


def _expand_mat_s():
    r = lax.broadcasted_iota(jnp.int32, (_FBS, _NPS), 0) // _K
    c = lax.broadcasted_iota(jnp.int32, (_FBS, _NPS), 1)
    return (r == c).astype(jnp.float32)


def _pe1stat_body(xg_ref, xl8_ref, pw1t_ref, pb1_ref, st_ref):
    e = _expand_mat_s()
    delta = xg_ref[:, 0:8] - jnp.dot(e, xl8_ref[...], preferred_element_type=jnp.float32)
    pe1 = jnp.dot(delta, pw1t_ref[...], preferred_element_type=jnp.float32) + pb1_ref[...]
    st_ref[0:1, 0:1, 0:_PH] = jnp.sum(pe1, axis=0, keepdims=True).reshape(1, 1, _PH)
    st_ref[0:1, 0:1, _PH:2 * _PH] = jnp.sum(pe1 * pe1, axis=0, keepdims=True).reshape(1, 1, _PH)


def _pe1stat(g, xl8, pw1t8, pb1):
    grid = (_NK // _FBS,)
    return pl.pallas_call(
        _pe1stat_body,
        grid=grid,
        in_specs=[
            pl.BlockSpec((_FBS, 128), lambda b: (b, 2)),
            pl.BlockSpec((_NPS, 8), lambda b: (b, 0)),
            pl.BlockSpec((8, _PH), lambda b: (0, 0)),
            pl.BlockSpec((1, _PH), lambda b: (0, 0)),
        ],
        out_specs=pl.BlockSpec((1, 1, 2 * _PH), lambda b: (b, 0, 0)),
        out_shape=jax.ShapeDtypeStruct((_NK // _FBS, 1, 2 * _PH), jnp.float32),
    )(g, xl8, pw1t8, pb1)


def _passB_body(xg_ref, kvg_ref, ql_ref, xl8_ref, pw1t_ref, pb1_ref,
                a1_ref, c1_ref, pw2t_ref, pb2_ref,
                wpre_ref, vout_ref, st_ref):
    u = lax.bitcast_convert_type(kvg_ref[...], jnp.int32)
    kg = lax.bitcast_convert_type(jnp.bitwise_and(u, jnp.int32(-65536)), jnp.float32)
    vg = lax.bitcast_convert_type(lax.shift_left(u, 16), jnp.float32)
    e = _expand_mat()
    delta = xg_ref[:, 0:8] - jnp.dot(e, xl8_ref[...], preferred_element_type=jnp.float32)
    pe1 = jnp.dot(delta, pw1t_ref[...], preferred_element_type=jnp.float32) + pb1_ref[...]
    pe1 = _leaky(pe1 * a1_ref[...] + c1_ref[...])
    pe2 = jnp.dot(pe1.astype(jnp.bfloat16), pw2t_ref[...].astype(jnp.bfloat16),
                  preferred_element_type=jnp.float32) + pb2_ref[...]
    qlr = jnp.dot(e, ql_ref[...], preferred_element_type=jnp.float32)
    wpre = qlr - kg + pe2
    wpre_ref[...] = wpre.astype(jnp.bfloat16)
    vout_ref[...] = (vg + pe2).astype(jnp.bfloat16)
    st_ref[0:1, 0:1, 0:_C] = jnp.sum(wpre, axis=0, keepdims=True).reshape(1, 1, _C)
    st_ref[0:1, 0:1, _C:2 * _C] = jnp.sum(wpre * wpre, axis=0, keepdims=True).reshape(1, 1, _C)


def _passB(g, ql, xl8, pw1t8, pb1, a1, c1, pw2t, pb2):
    grid = (_NK // _FB,)
    return pl.pallas_call(
        _passB_body,
        grid=grid,
        in_specs=[
            pl.BlockSpec((_FB, 128), lambda b: (b, 2)),
            pl.BlockSpec((_FB, _C), lambda b: (b, 0)),
            pl.BlockSpec((_NP, _C), lambda b: (b, 0)),
            pl.BlockSpec((_NP, 8), lambda b: (b, 0)),
            pl.BlockSpec((8, _PH), lambda b: (0, 0)),
            pl.BlockSpec((1, _PH), lambda b: (0, 0)),
            pl.BlockSpec((1, _PH), lambda b: (0, 0)),
            pl.BlockSpec((1, _PH), lambda b: (0, 0)),
            pl.BlockSpec((_PH, _C), lambda b: (0, 0)),
            pl.BlockSpec((1, _C), lambda b: (0, 0)),
        ],
        out_specs=[
            pl.BlockSpec((_FB, _C), lambda b: (b, 0)),
            pl.BlockSpec((_FB, _C), lambda b: (b, 0)),
            pl.BlockSpec((1, 1, 2 * _C), lambda b: (b, 0, 0)),
        ],
        out_shape=[
            jax.ShapeDtypeStruct((_NK, _C), jnp.bfloat16),
            jax.ShapeDtypeStruct((_NK, _C), jnp.bfloat16),
            jax.ShapeDtypeStruct((_NK // _FB, 1, 2 * _C), jnp.float32),
        ],
    )(g, g, ql, xl8, pw1t8, pb1, a1, c1, pw2t, pb2)


def _passC_body(wpre_ref, ag_ref, cg_ref, wwt_ref, wb_ref, h_ref, st_ref):
    s = _leaky(wpre_ref[...].astype(jnp.float32) * ag_ref[...] + cg_ref[...])
    h = jnp.dot(s.astype(jnp.bfloat16), wwt_ref[...].astype(jnp.bfloat16),
                preferred_element_type=jnp.float32) + wb_ref[...]
    h_ref[...] = h.astype(jnp.bfloat16)
    st_ref[0:1, 0:1, 0:_C] = jnp.sum(h, axis=0, keepdims=True).reshape(1, 1, _C)
    st_ref[0:1, 0:1, _C:2 * _C] = jnp.sum(h * h, axis=0, keepdims=True).reshape(1, 1, _C)


def _passC(wpre, ag1, cg1, wwt, wb):
    grid = (_NK // _FB,)
    return pl.pallas_call(
        _passC_body,
        grid=grid,
        in_specs=[
            pl.BlockSpec((_FB, _C), lambda b: (b, 0)),
            pl.BlockSpec((1, _C), lambda b: (0, 0)),
            pl.BlockSpec((1, _C), lambda b: (0, 0)),
            pl.BlockSpec((_C, _C), lambda b: (0, 0)),
            pl.BlockSpec((1, _C), lambda b: (0, 0)),
        ],
        out_specs=[
            pl.BlockSpec((_FB, _C), lambda b: (b, 0)),
            pl.BlockSpec((1, 1, 2 * _C), lambda b: (b, 0, 0)),
        ],
        out_shape=[
            jax.ShapeDtypeStruct((_NK, _C), jnp.bfloat16),
            jax.ShapeDtypeStruct((_NK // _FB, 1, 2 * _C), jnp.float32),
        ],
    )(wpre, ag1, cg1, wwt, wb)


def _tree_red(x, op):
    w = _K
    while w > 1:
        h = w // 2
        x = op(x[:, 0:h], x[:, h:w])
        w = h
    return x


def _passD_body(h_ref, v_ref, ag_ref, cg_ref, out_ref):
    s = _leaky(h_ref[...].astype(jnp.float32) * ag_ref[...] + cg_ref[...])
    m = _tree_red(s, jnp.maximum)
    ex = jnp.exp(s - m)
    den = _tree_red(ex, jnp.add)
    w = ex / den
    out_ref[...] = _tree_red(w * v_ref[...].astype(jnp.float32), jnp.add)


_NPD = 64


def _passD(h3, v3, ag2, cg2):
    grid = (_N // _NPD,)
    return pl.pallas_call(
        _passD_body,
        grid=grid,
        in_specs=[
            pl.BlockSpec((_NPD, _K, _C), lambda b: (b, 0, 0)),
            pl.BlockSpec((_NPD, _K, _C), lambda b: (b, 0, 0)),
            pl.BlockSpec((1, 1, _C), lambda b: (0, 0, 0)),
            pl.BlockSpec((1, 1, _C), lambda b: (0, 0, 0)),
        ],
        out_specs=pl.BlockSpec((_NPD, 1, _C), lambda b: (b, 0, 0)),
        out_shape=jax.ShapeDtypeStruct((_N, 1, _C), jnp.float32),
    )(h3, v3, ag2, cg2)


def _bn_ab(stats3, width, gamma, beta):
    stats = stats3.reshape(-1, 2 * width)
    s1 = jnp.sum(stats[:, 0:width], axis=0)
    s2 = jnp.sum(stats[:, width:2 * width], axis=0)
    mean = s1 / _NK
    var = s2 / _NK - mean * mean
    a = gamma / jnp.sqrt(var + _EPS)
    b = beta - mean * a
    return a.reshape(1, width), b.reshape(1, width)


def kernel(fea_i, fea_last, xyz_i, xyz_last, batch, t_i,
           p_w1, p_b1, p_g1, p_be1, p_w2, p_b2,
           q_w, q_b, k_w, k_b, v_w, v_b,
           w_g1, w_be1, w_w, w_b, w_g2, w_be2):
    f32 = jnp.float32
    t_i = jnp.asarray(t_i, f32)

    ql, kv = _proj(fea_last, fea_i,
                   q_w.T, q_b.reshape(1, _C), k_w.T, k_b.reshape(1, _C),
                   v_w.T, v_b.reshape(1, _C))

    byr = batch.reshape(1, _N)
    bxc = batch.reshape(_N, 1)
    yt8 = jnp.concatenate([xyz_last.T, jnp.zeros((5, _N), f32)], axis=0)
    x8r = jnp.concatenate([xyz_i, jnp.zeros((_N, 5), f32)], axis=1)
    bounds = jnp.searchsorted(batch, jnp.arange(9, dtype=jnp.int32),
                              side="left").astype(jnp.int32)
    bf = batch[0::_RB]
    bl = batch[_RB - 1::_RB]
    tlo = (bounds[bf] // _CT).astype(jnp.int32)
    thi = ((bounds[bl + 1] + _CT - 1) // _CT).astype(jnp.int32)
    idx = _knn(yt8, byr, x8r, bxc, tlo, thi)
    idx_flat = idx.T.reshape(_NK)

    table = jnp.concatenate(
        [kv, xyz_i, jnp.full((_N, 1), t_i, f32), jnp.zeros((_N, 124), f32)], axis=1)
    g = _gather(table, idx_flat)

    xl8 = jnp.concatenate(
        [xyz_last, jnp.full((_N, 1), _T_LAST, f32), jnp.zeros((_N, 4), f32)], axis=1)
    pw1t8 = jnp.concatenate([p_w1.T, jnp.zeros((4, _PH), f32)], axis=0)
    pb1 = p_b1.reshape(1, _PH)

    st1 = _pe1stat(g, xl8, pw1t8, pb1)
    a1, c1 = _bn_ab(st1, _PH, p_g1, p_be1)

    wpre, vout, st2 = _passB(g, ql, xl8, pw1t8, pb1, a1, c1,
                             p_w2.T, p_b2.reshape(1, _C))
    ag1, cg1 = _bn_ab(st2, _C, w_g1, w_be1)

    h, st3 = _passC(wpre, ag1, cg1, w_w.T, w_b.reshape(1, _C))
    ag2, cg2 = _bn_ab(st3, _C, w_g2, w_be2)

    out3 = _passD(h.reshape(_N, _K, _C), vout.reshape(_N, _K, _C),
                  ag2.reshape(1, 1, _C), cg2.reshape(1, 1, _C))
    return out3.reshape(_N, _C)
